# Initial kernel scaffold; baseline (speedup 1.0000x reference)
import jax
import jax.numpy as jnp
from jax.experimental import pallas as pl


def kernel(x, edge_index, W1_self, W1_neigh, b1, W2_self, W2_neigh, b2):
    raise NotImplementedError("write your pallas kernel here")


# speedup vs baseline: 8.6583x; 8.6583x over previous
import functools

import jax
import jax.numpy as jnp
from jax import lax
from jax.experimental import pallas as pl
from jax.experimental.pallas import tpu as pltpu
from jax.experimental.pallas import tpu_sc as plsc

N_NODES = 10000
N_EDGES = 320000
D = 128

NC = 2
NS = 16
NW = NC * NS

WIN = 128
N_WINDOWS = 2560
WPW = N_WINDOWS // NW
E_PAD = N_WINDOWS * WIN
N_PAD = 10240
RPT = N_PAD // NS
Z_ROWS = 32


def _sc_agg_body(compute_cnt, x_hbm, src_hbm, dst_hbm, *refs):
    if compute_cnt:
        (agg_out, cnt_out, sidx, didx, rows, zbuf, zcnt, ones_v,
         agg_sh, cnt_sh, gsem) = refs
    else:
        agg_out, sidx, didx, rows, zbuf, agg_sh, gsem = refs

    c = lax.axis_index("c")
    s = lax.axis_index("s")
    wid = s * NC + c

    pltpu.sync_copy(src_hbm.at[pl.ds(wid * WPW, WPW)], sidx)
    pltpu.sync_copy(dst_hbm.at[pl.ds(wid * WPW, WPW)], didx)

    z16 = jnp.zeros((16,), jnp.float32)

    def _zrow(i, carry):
        for j in range(D // 16):
            zbuf[i, pl.ds(j * 16, 16)] = z16
        return carry

    lax.fori_loop(0, Z_ROWS, _zrow, 0)

    base = s * RPT
    for t in range(RPT // Z_ROWS):
        pltpu.sync_copy(zbuf, agg_sh.at[pl.ds(base + t * Z_ROWS, Z_ROWS)])

    if compute_cnt:
        one16 = jnp.ones((16,), jnp.float32)
        for j in range(WIN // 16):
            ones_v[pl.ds(j * 16, 16)] = one16

        def _zc(i, carry):
            zcnt[pl.ds(i * 16, 16)] = z16
            return carry

        lax.fori_loop(0, RPT // 16, _zc, 0)
        pltpu.sync_copy(zcnt, cnt_sh.at[pl.ds(base, RPT)])

    plsc.subcore_barrier()

    def _step(w, carry):
        pltpu.async_copy(x_hbm.at[sidx.at[w]], rows, gsem).wait()
        pltpu.sync_copy(rows, agg_sh.at[didx.at[w]], add=True)
        if compute_cnt:
            pltpu.sync_copy(ones_v, cnt_sh.at[didx.at[w]], add=True)
        return carry

    lax.fori_loop(0, WPW, _step, 0)

    plsc.subcore_barrier()

    for t in range(RPT // WIN):
        pltpu.sync_copy(agg_sh.at[pl.ds(base + t * WIN, WIN)],
                        agg_out.at[c, pl.ds(base + t * WIN, WIN)])
    if compute_cnt:
        pltpu.sync_copy(cnt_sh.at[pl.ds(base, RPT)],
                        cnt_out.at[c, pl.ds(base, RPT)])


def _make_sc_agg(compute_cnt):
    mesh = plsc.VectorSubcoreMesh(core_axis_name="c", subcore_axis_name="s",
                                  num_cores=NC, num_subcores=NS)
    out_type = [jax.ShapeDtypeStruct((NC, N_PAD, D), jnp.float32)]
    scratch = [
        pltpu.VMEM((WPW, WIN), jnp.int32),
        pltpu.VMEM((WPW, WIN), jnp.int32),
        pltpu.VMEM((WIN, D), jnp.float32),
        pltpu.VMEM((Z_ROWS, D), jnp.float32),
    ]
    if compute_cnt:
        out_type.append(jax.ShapeDtypeStruct((NC, N_PAD), jnp.float32))
        scratch += [
            pltpu.VMEM((RPT,), jnp.float32),
            pltpu.VMEM((WIN,), jnp.float32),
        ]
    scratch.append(pltpu.VMEM_SHARED((N_PAD, D), jnp.float32))
    if compute_cnt:
        scratch.append(pltpu.VMEM_SHARED((N_PAD,), jnp.float32))
    scratch.append(pltpu.SemaphoreType.DMA)

    return pl.kernel(
        functools.partial(_sc_agg_body, compute_cnt),
        out_type=tuple(out_type),
        mesh=mesh,
        scratch_types=scratch,
        name=f"sage_sc_agg_cnt{int(compute_cnt)}",
    )


_SC_AGG_CNT = _make_sc_agg(True)
_SC_AGG = _make_sc_agg(False)


def _combine_body(relu, a0, a1, c0, c1, xr, wn, ws, br, o):
    cnt = c0[...] + c1[...]
    r = 1.0 / jnp.maximum(cnt, 1.0)
    mean = (a0[...] + a1[...]) * r
    acc = jnp.dot(mean, wn[...], preferred_element_type=jnp.float32)
    acc = acc + jnp.dot(xr[...], ws[...], preferred_element_type=jnp.float32)
    acc = acc + br[...]
    if relu:
        acc = jnp.maximum(acc, 0.0)
    o[...] = acc


_R_BLK = 1024


def _combine(a0, a1, c0, c1, xr, wn, ws, br, relu):
    grid = (N_PAD // _R_BLK,)
    row_spec = pl.BlockSpec((_R_BLK, D), lambda i: (i, 0))
    cnt_spec = pl.BlockSpec((_R_BLK, 1), lambda i: (i, 0))
    w_spec = pl.BlockSpec((D, D), lambda i: (0, 0))
    b_spec = pl.BlockSpec((1, D), lambda i: (0, 0))
    return pl.pallas_call(
        functools.partial(_combine_body, relu),
        grid=grid,
        in_specs=[row_spec, row_spec, cnt_spec, cnt_spec, row_spec,
                  w_spec, w_spec, b_spec],
        out_specs=row_spec,
        out_shape=jax.ShapeDtypeStruct((N_PAD, D), jnp.float32),
        name=f"sage_combine_relu{int(relu)}",
    )(a0, a1, c0, c1, xr, wn, ws, br)


def kernel(x, edge_index, W1_self, W1_neigh, b1, W2_self, W2_neigh, b2):
    src = edge_index[0]
    dst = edge_index[1]
    pad = E_PAD - N_EDGES
    pad_ids = jnp.arange(pad, dtype=jnp.int32)
    src_p = jnp.concatenate([src, pad_ids % N_NODES]).reshape(N_WINDOWS, WIN)
    dst_p = jnp.concatenate([dst, N_NODES + pad_ids % (N_PAD - N_NODES)]
                            ).reshape(N_WINDOWS, WIN)
    xp = jnp.pad(x, ((0, N_PAD - N_NODES), (0, 0)))

    agg1, cnt = _SC_AGG_CNT(xp, src_p, dst_p)
    c0 = cnt[0].reshape(N_PAD, 1)
    c1 = cnt[1].reshape(N_PAD, 1)
    b1r = b1.reshape(1, D)
    b2r = b2.reshape(1, D)

    h = _combine(agg1[0], agg1[1], c0, c1, xp, W1_neigh, W1_self, b1r,
                 relu=True)
    (agg2,) = _SC_AGG(h, src_p, dst_p)
    out = _combine(agg2[0], agg2[1], c0, c1, h, W2_neigh, W2_self, b2r,
                   relu=False)
    return out[:N_NODES]


# speedup vs baseline: 10.8781x; 1.2564x over previous
import functools

import jax
import jax.numpy as jnp
from jax import lax
from jax.experimental import pallas as pl
from jax.experimental.pallas import tpu as pltpu
from jax.experimental.pallas import tpu_sc as plsc

N_NODES = 10000
N_EDGES = 320000
D = 128

NC = 2
NS = 16
NW = NC * NS

WIN = 128
G = 8
N_WINDOWS = 2560
WPW = N_WINDOWS // NW
NG = WPW // G
IDX_WINDOWS = N_WINDOWS + G
E_PAD = IDX_WINDOWS * WIN
N_PAD = 10240
RPT = N_PAD // NS
Z_ROWS = 32


def _sc_agg_body(compute_cnt, x_hbm, src_hbm, dst_hbm, *refs):
    if compute_cnt:
        (agg_out, cnt_out, sidx0, sidx1, didx0, didx1, rows0, rows1,
         pidx, pidx_lo, zbuf, zcnt, ones_v, agg_sh, cnt_sh,
         gsem0, gsem1, ssem0, ssem1, csem0, csem1, isem0, isem1) = refs
    else:
        (agg_out, sidx0, sidx1, didx0, didx1, rows0, rows1,
         pidx, pidx_lo, zbuf, agg_sh,
         gsem0, gsem1, ssem0, ssem1, isem0, isem1) = refs

    sidx = (sidx0, sidx1)
    didx = (didx0, didx1)
    rows = (rows0, rows1)
    gsem = (gsem0, gsem1)
    ssem = (ssem0, ssem1)

    c = lax.axis_index("c")
    s = lax.axis_index("s")
    wid = s * NC + c
    wbase = wid * WPW

    z16 = jnp.zeros((16,), jnp.float32)
    iota16 = lax.iota(jnp.int32, 16)
    for j in range(WIN // 16):
        pidx[pl.ds(j * 16, 16)] = iota16 + (
            N_NODES + (s * WIN + j * 16) % (N_PAD - N_NODES - 16))
        pidx_lo[pl.ds(j * 16, 16)] = iota16 + s * WIN + j * 16

    def _zrow(i, carry):
        for j in range(D // 16):
            zbuf[i, pl.ds(j * 16, 16)] = z16
        return carry

    lax.fori_loop(0, Z_ROWS, _zrow, 0)

    if compute_cnt:
        csem = (csem0, csem1)
        one16 = jnp.ones((16,), jnp.float32)
        for j in range(WIN // 16):
            ones_v[pl.ds(j * 16, 16)] = one16

        def _zc(i, carry):
            zcnt[pl.ds(i * 16, 16)] = z16
            return carry

        lax.fori_loop(0, RPT // 16, _zc, 0)

    base = s * RPT
    for t in range(RPT // Z_ROWS):
        pltpu.sync_copy(zbuf, agg_sh.at[pl.ds(base + t * Z_ROWS, Z_ROWS)])
    if compute_cnt:
        pltpu.sync_copy(zcnt, cnt_sh.at[pl.ds(base, RPT)])
    plsc.subcore_barrier()

    def wait_gather(b):
        pltpu.make_async_copy(x_hbm.at[pidx_lo], rows[b], gsem[b]).wait()

    def wait_scatter(b):
        pltpu.make_async_copy(rows[b], agg_sh.at[pidx], ssem[b]).wait()
        if compute_cnt:
            pltpu.make_async_copy(ones_v, cnt_sh.at[pidx], csem[b]).wait()

    def wait_idx(gb, off):
        pltpu.make_async_copy(src_hbm.at[pl.ds(off, G)], sidx[gb],
                              isem[gb]).wait()
        pltpu.make_async_copy(dst_hbm.at[pl.ds(off, G)], didx[gb],
                              isem[gb]).wait()

    isem = (isem0, isem1)

    pltpu.async_copy(src_hbm.at[pl.ds(wbase, G)], sidx0, isem0)
    pltpu.async_copy(dst_hbm.at[pl.ds(wbase, G)], didx0, isem0)
    for j in range(D // 16):
        didx1[G - 1, pl.ds(j * 16, 16)] = pidx[pl.ds(j * 16, 16)]
    pltpu.async_copy(x_hbm.at[pidx_lo], rows1, gsem1)
    for b in range(2):
        pltpu.async_copy(rows[b], agg_sh.at[pidx], ssem[b], add=True)
        if compute_cnt:
            pltpu.async_copy(ones_v, cnt_sh.at[pidx], csem[b], add=True)

    def pair_body(gg, carry):
        for gpar in range(2):
            gb = gpar
            g = 2 * gg + gpar
            gwbase = wbase + g * G
            wait_idx(gb, gwbase)
            for k in range(G):
                b = k % 2
                bv = (k + 1) % 2
                wait_scatter(b)
                pltpu.async_copy(x_hbm.at[sidx[gb].at[k]], rows[b], gsem[b])
                wait_gather(bv)
                dv = didx[gb ^ 1].at[G - 1] if k == 0 else didx[gb].at[k - 1]
                pltpu.async_copy(rows[bv], agg_sh.at[dv], ssem[bv], add=True)
                if compute_cnt:
                    pltpu.async_copy(ones_v, cnt_sh.at[dv], csem[bv],
                                     add=True)
                if k == 1:
                    nxt = wbase + (g + 1) * G
                    pltpu.async_copy(src_hbm.at[pl.ds(nxt, G)], sidx[gb ^ 1],
                                     isem[gb ^ 1])
                    pltpu.async_copy(dst_hbm.at[pl.ds(nxt, G)], didx[gb ^ 1],
                                     isem[gb ^ 1])
        return carry

    lax.fori_loop(0, NG // 2, pair_body, 0)

    wait_gather(1)
    pltpu.async_copy(rows1, agg_sh.at[didx1.at[G - 1]], ssem1, add=True)
    if compute_cnt:
        pltpu.async_copy(ones_v, cnt_sh.at[didx1.at[G - 1]], csem1, add=True)
    wait_scatter(0)
    wait_scatter(1)
    wait_scatter(1)
    wait_idx(0, wbase + NG * G)

    plsc.subcore_barrier()

    for t in range(RPT // WIN):
        pltpu.sync_copy(agg_sh.at[pl.ds(base + t * WIN, WIN)],
                        agg_out.at[c, pl.ds(base + t * WIN, WIN)])
    if compute_cnt:
        pltpu.sync_copy(cnt_sh.at[pl.ds(base, RPT)],
                        cnt_out.at[c, pl.ds(base, RPT)])


def _make_sc_agg(compute_cnt):
    mesh = plsc.VectorSubcoreMesh(core_axis_name="c", subcore_axis_name="s",
                                  num_cores=NC, num_subcores=NS)
    out_type = [jax.ShapeDtypeStruct((NC, N_PAD, D), jnp.float32)]
    if compute_cnt:
        out_type.append(jax.ShapeDtypeStruct((NC, N_PAD), jnp.float32))
    scratch = [
        pltpu.VMEM((G, WIN), jnp.int32),
        pltpu.VMEM((G, WIN), jnp.int32),
        pltpu.VMEM((G, WIN), jnp.int32),
        pltpu.VMEM((G, WIN), jnp.int32),
        pltpu.VMEM((WIN, D), jnp.float32),
        pltpu.VMEM((WIN, D), jnp.float32),
        pltpu.VMEM((WIN,), jnp.int32),
        pltpu.VMEM((WIN,), jnp.int32),
        pltpu.VMEM((Z_ROWS, D), jnp.float32),
    ]
    if compute_cnt:
        scratch += [
            pltpu.VMEM((RPT,), jnp.float32),
            pltpu.VMEM((WIN,), jnp.float32),
        ]
    scratch.append(pltpu.VMEM_SHARED((N_PAD, D), jnp.float32))
    if compute_cnt:
        scratch.append(pltpu.VMEM_SHARED((N_PAD,), jnp.float32))
    nsem = 6 + (2 if compute_cnt else 0)
    scratch += [pltpu.SemaphoreType.DMA] * nsem

    return pl.kernel(
        functools.partial(_sc_agg_body, compute_cnt),
        out_type=tuple(out_type),
        mesh=mesh,
        scratch_types=scratch,
        name=f"sage_sc_agg_cnt{int(compute_cnt)}",
    )


_SC_AGG_CNT = _make_sc_agg(True)
_SC_AGG = _make_sc_agg(False)


def _combine_body(relu, a0, a1, c0, c1, xr, wn, ws, br, o):
    cnt = c0[...] + c1[...]
    r = 1.0 / jnp.maximum(cnt, 1.0)
    mean = (a0[...] + a1[...]) * r
    acc = jnp.dot(mean, wn[...], preferred_element_type=jnp.float32)
    acc = acc + jnp.dot(xr[...], ws[...], preferred_element_type=jnp.float32)
    acc = acc + br[...]
    if relu:
        acc = jnp.maximum(acc, 0.0)
    o[...] = acc


_R_BLK = 400


def _combine(a0, a1, c0, c1, xr, wn, ws, br, relu):
    grid = (N_NODES // _R_BLK,)
    row_spec = pl.BlockSpec((_R_BLK, D), lambda i: (i, 0))
    cnt_spec = pl.BlockSpec((_R_BLK, 1), lambda i: (i, 0))
    w_spec = pl.BlockSpec((D, D), lambda i: (0, 0))
    b_spec = pl.BlockSpec((1, D), lambda i: (0, 0))
    return pl.pallas_call(
        functools.partial(_combine_body, relu),
        grid=grid,
        in_specs=[row_spec, row_spec, cnt_spec, cnt_spec, row_spec,
                  w_spec, w_spec, b_spec],
        out_specs=row_spec,
        out_shape=jax.ShapeDtypeStruct((N_NODES, D), jnp.float32),
        name=f"sage_combine_relu{int(relu)}",
    )(a0, a1, c0, c1, xr, wn, ws, br)


def kernel(x, edge_index, W1_self, W1_neigh, b1, W2_self, W2_neigh, b2):
    src = edge_index[0]
    dst = edge_index[1]
    pad = E_PAD - N_EDGES
    pad_ids = jnp.arange(pad, dtype=jnp.int32)
    src_p = jnp.concatenate([src, pad_ids % N_NODES]).reshape(IDX_WINDOWS, WIN)
    dst_p = jnp.concatenate([dst, N_NODES + pad_ids % (N_PAD - N_NODES)]
                            ).reshape(IDX_WINDOWS, WIN)

    agg1, cnt = _SC_AGG_CNT(x, src_p, dst_p)
    c0 = cnt[0].reshape(N_PAD, 1)
    c1 = cnt[1].reshape(N_PAD, 1)
    b1r = b1.reshape(1, D)
    b2r = b2.reshape(1, D)

    h = _combine(agg1[0], agg1[1], c0, c1, x, W1_neigh, W1_self, b1r,
                 relu=True)
    (agg2,) = _SC_AGG(h, src_p, dst_p)
    out = _combine(agg2[0], agg2[1], c0, c1, h, W2_neigh, W2_self, b2r,
                   relu=False)
    return out


# speedup vs baseline: 11.5819x; 1.0647x over previous
import functools

import jax
import jax.numpy as jnp
from jax import lax
from jax.experimental import pallas as pl
from jax.experimental.pallas import tpu as pltpu
from jax.experimental.pallas import tpu_sc as plsc

N_NODES = 10000
N_EDGES = 320000
D = 128

NC = 2
NS = 16
NW = NC * NS

WIN = 128
G = 8
N_WINDOWS = 2560
WPW = N_WINDOWS // NW
NG = WPW // G
IDX_WINDOWS = N_WINDOWS + G
E_PAD = IDX_WINDOWS * WIN
N_PAD = 10240
RPT = N_PAD // NS
Z_ROWS = 32


def _sc_agg_body(compute_cnt, x_hbm, src_hbm, dst_hbm, *refs):
    if compute_cnt:
        (agg_out, cnt_out, sidx0, sidx1, didx0, didx1, rows0, rows1,
         pidx, pidx_lo, zbuf, zcnt, ones_v, agg_sh, cnt_sh,
         gsem0, gsem1, ssem0, ssem1, csem0, csem1, isem0, isem1) = refs
    else:
        (agg_out, sidx0, sidx1, didx0, didx1, rows0, rows1,
         pidx, pidx_lo, zbuf, agg_sh,
         gsem0, gsem1, ssem0, ssem1, isem0, isem1) = refs

    sidx = (sidx0, sidx1)
    didx = (didx0, didx1)
    rows = (rows0, rows1)
    gsem = (gsem0, gsem1)
    ssem = (ssem0, ssem1)

    c = lax.axis_index("c")
    s = lax.axis_index("s")
    wid = s * NC + c
    wbase = wid * WPW

    z16 = jnp.zeros((16,), jnp.float32)
    iota16 = lax.iota(jnp.int32, 16)
    for j in range(WIN // 16):
        pidx[pl.ds(j * 16, 16)] = iota16 + (
            N_NODES + (s * WIN + j * 16) % (N_PAD - N_NODES - 16))
        pidx_lo[pl.ds(j * 16, 16)] = iota16 + s * WIN + j * 16

    def _zrow(i, carry):
        for j in range(D // 16):
            zbuf[i, pl.ds(j * 16, 16)] = z16
        return carry

    lax.fori_loop(0, Z_ROWS, _zrow, 0)

    if compute_cnt:
        csem = (csem0, csem1)
        one16 = jnp.ones((16,), jnp.float32)
        for j in range(WIN // 16):
            ones_v[pl.ds(j * 16, 16)] = one16

        def _zc(i, carry):
            zcnt[pl.ds(i * 16, 16)] = z16
            return carry

        lax.fori_loop(0, RPT // 16, _zc, 0)

    base = s * RPT
    for t in range(RPT // Z_ROWS):
        pltpu.sync_copy(zbuf, agg_sh.at[pl.ds(base + t * Z_ROWS, Z_ROWS)])
    if compute_cnt:
        pltpu.sync_copy(zcnt, cnt_sh.at[pl.ds(base, RPT)])
    plsc.subcore_barrier()

    def wait_gather(b):
        pltpu.make_async_copy(x_hbm.at[pidx_lo], rows[b], gsem[b]).wait()

    def wait_scatter(b):
        pltpu.make_async_copy(rows[b], agg_sh.at[pidx], ssem[b]).wait()
        if compute_cnt:
            pltpu.make_async_copy(ones_v, cnt_sh.at[pidx], csem[b]).wait()

    def wait_idx(gb, off):
        pltpu.make_async_copy(src_hbm.at[pl.ds(off, G)], sidx[gb],
                              isem[gb]).wait()
        pltpu.make_async_copy(dst_hbm.at[pl.ds(off, G)], didx[gb],
                              isem[gb]).wait()

    isem = (isem0, isem1)

    pltpu.async_copy(src_hbm.at[pl.ds(wbase, G)], sidx0, isem0)
    pltpu.async_copy(dst_hbm.at[pl.ds(wbase, G)], didx0, isem0)
    for j in range(D // 16):
        didx1[G - 1, pl.ds(j * 16, 16)] = pidx[pl.ds(j * 16, 16)]
    pltpu.async_copy(x_hbm.at[pidx_lo], rows1, gsem1)
    for b in range(2):
        pltpu.async_copy(rows[b], agg_sh.at[pidx], ssem[b], add=True)
        if compute_cnt:
            pltpu.async_copy(ones_v, cnt_sh.at[pidx], csem[b], add=True)

    def pair_body(gg, carry):
        for gpar in range(2):
            gb = gpar
            g = 2 * gg + gpar
            gwbase = wbase + g * G
            wait_idx(gb, gwbase)
            for k in range(G):
                b = k % 2
                bv = (k + 1) % 2
                wait_scatter(b)
                pltpu.async_copy(x_hbm.at[sidx[gb].at[k]], rows[b], gsem[b])
                wait_gather(bv)
                dv = didx[gb ^ 1].at[G - 1] if k == 0 else didx[gb].at[k - 1]
                pltpu.async_copy(rows[bv], agg_sh.at[dv], ssem[bv], add=True)
                if compute_cnt:
                    pltpu.async_copy(ones_v, cnt_sh.at[dv], csem[bv],
                                     add=True)
                if k == 1:
                    nxt = wbase + (g + 1) * G
                    pltpu.async_copy(src_hbm.at[pl.ds(nxt, G)], sidx[gb ^ 1],
                                     isem[gb ^ 1])
                    pltpu.async_copy(dst_hbm.at[pl.ds(nxt, G)], didx[gb ^ 1],
                                     isem[gb ^ 1])
        return carry

    lax.fori_loop(0, NG // 2, pair_body, 0)

    wait_gather(1)
    pltpu.async_copy(rows1, agg_sh.at[didx1.at[G - 1]], ssem1, add=True)
    if compute_cnt:
        pltpu.async_copy(ones_v, cnt_sh.at[didx1.at[G - 1]], csem1, add=True)
    wait_scatter(0)
    wait_scatter(1)
    wait_scatter(1)
    wait_idx(0, wbase + NG * G)

    plsc.subcore_barrier()

    for t in range(RPT // WIN):
        pltpu.sync_copy(agg_sh.at[pl.ds(base + t * WIN, WIN)],
                        agg_out.at[c, pl.ds(base + t * WIN, WIN)])
    if compute_cnt:
        pltpu.sync_copy(cnt_sh.at[pl.ds(base, RPT)],
                        cnt_out.at[c, pl.ds(base, RPT)])


def _make_sc_agg(compute_cnt):
    mesh = plsc.VectorSubcoreMesh(core_axis_name="c", subcore_axis_name="s",
                                  num_cores=NC, num_subcores=NS)
    out_type = [jax.ShapeDtypeStruct((NC, N_PAD, D), jnp.float32)]
    if compute_cnt:
        out_type.append(jax.ShapeDtypeStruct((NC, N_PAD), jnp.float32))
    scratch = [
        pltpu.VMEM((G, WIN), jnp.int32),
        pltpu.VMEM((G, WIN), jnp.int32),
        pltpu.VMEM((G, WIN), jnp.int32),
        pltpu.VMEM((G, WIN), jnp.int32),
        pltpu.VMEM((WIN, D), jnp.float32),
        pltpu.VMEM((WIN, D), jnp.float32),
        pltpu.VMEM((WIN,), jnp.int32),
        pltpu.VMEM((WIN,), jnp.int32),
        pltpu.VMEM((Z_ROWS, D), jnp.float32),
    ]
    if compute_cnt:
        scratch += [
            pltpu.VMEM((RPT,), jnp.float32),
            pltpu.VMEM((WIN,), jnp.float32),
        ]
    scratch.append(pltpu.VMEM_SHARED((N_PAD, D), jnp.float32))
    if compute_cnt:
        scratch.append(pltpu.VMEM_SHARED((N_PAD,), jnp.float32))
    nsem = 6 + (2 if compute_cnt else 0)
    scratch += [pltpu.SemaphoreType.DMA] * nsem

    return pl.kernel(
        functools.partial(_sc_agg_body, compute_cnt),
        out_type=tuple(out_type),
        mesh=mesh,
        scratch_types=scratch,
        name=f"sage_sc_agg_cnt{int(compute_cnt)}",
    )


_SC_AGG_CNT = _make_sc_agg(True)
_SC_AGG = _make_sc_agg(False)


def _combine_body(relu, a0, a1, c0, c1, xr, wn, ws, br, o):
    cnt = c0[...] + c1[...]
    r = 1.0 / jnp.maximum(cnt, 1.0)
    mean = (a0[...] + a1[...]) * r
    acc = jnp.dot(mean, wn[...], preferred_element_type=jnp.float32)
    acc = acc + jnp.dot(xr[...], ws[...], preferred_element_type=jnp.float32)
    acc = acc + br[...]
    if relu:
        acc = jnp.maximum(acc, 0.0)
    o[...] = acc


_R_BLK = 2000


def _combine(a0, a1, c0, c1, xr, wn, ws, br, relu):
    grid = (N_NODES // _R_BLK,)
    row_spec = pl.BlockSpec((_R_BLK, D), lambda i: (i, 0))
    cnt_spec = pl.BlockSpec((_R_BLK, 1), lambda i: (i, 0))
    w_spec = pl.BlockSpec((D, D), lambda i: (0, 0))
    b_spec = pl.BlockSpec((1, D), lambda i: (0, 0))
    return pl.pallas_call(
        functools.partial(_combine_body, relu),
        grid=grid,
        in_specs=[row_spec, row_spec, cnt_spec, cnt_spec, row_spec,
                  w_spec, w_spec, b_spec],
        out_specs=row_spec,
        out_shape=jax.ShapeDtypeStruct((N_NODES, D), jnp.float32),
        name=f"sage_combine_relu{int(relu)}",
    )(a0, a1, c0, c1, xr, wn, ws, br)


def kernel(x, edge_index, W1_self, W1_neigh, b1, W2_self, W2_neigh, b2):
    src = edge_index[0]
    dst = edge_index[1]
    pad = E_PAD - N_EDGES
    pad_ids = jnp.arange(pad, dtype=jnp.int32)
    src_p = jnp.concatenate([src, pad_ids % N_NODES]).reshape(IDX_WINDOWS, WIN)
    dst_p = jnp.concatenate([dst, N_NODES + pad_ids % (N_PAD - N_NODES)]
                            ).reshape(IDX_WINDOWS, WIN)

    agg1, cnt = _SC_AGG_CNT(x, src_p, dst_p)
    c0 = cnt[0].reshape(N_PAD, 1)
    c1 = cnt[1].reshape(N_PAD, 1)
    b1r = b1.reshape(1, D)
    b2r = b2.reshape(1, D)

    h = _combine(agg1[0], agg1[1], c0, c1, x, W1_neigh, W1_self, b1r,
                 relu=True)
    (agg2,) = _SC_AGG(h, src_p, dst_p)
    out = _combine(agg2[0], agg2[1], c0, c1, h, W2_neigh, W2_self, b2r,
                   relu=False)
    return out


# speedup vs baseline: 13.5902x; 1.1734x over previous
import functools

import jax
import jax.numpy as jnp
from jax import lax
from jax.experimental import pallas as pl
from jax.experimental.pallas import tpu as pltpu
from jax.experimental.pallas import tpu_sc as plsc

N_NODES = 10000
N_EDGES = 320000
D = 128

NC = 2
NS = 16
NW = NC * NS

WIN = 64
G = 8
N_WINDOWS = 5120
WPW = N_WINDOWS // NW
NG = WPW // G
IDX_WINDOWS = N_WINDOWS + G
E_PAD = IDX_WINDOWS * WIN
N_PAD = 10240
RPT = N_PAD // NS
NBUF = 4
LAG = 2


def _sc_agg_body(compute_cnt, x_hbm, src_hbm, dst_hbm, *refs):
    if compute_cnt:
        (agg_out, cnt_out, sidx0, sidx1, didx0, didx1,
         rows0, rows1, rows2, rows3, pidx, pidx_lo, ones_v,
         agg_sh, cnt_sh,
         gsem0, gsem1, gsem2, gsem3, ssem0, ssem1, ssem2, ssem3,
         csem0, csem1, csem2, csem3, isem0, isem1) = refs
        csem = (csem0, csem1, csem2, csem3)
    else:
        (agg_out, sidx0, sidx1, didx0, didx1,
         rows0, rows1, rows2, rows3, pidx, pidx_lo,
         agg_sh,
         gsem0, gsem1, gsem2, gsem3, ssem0, ssem1, ssem2, ssem3,
         isem0, isem1) = refs

    sidx = (sidx0, sidx1)
    didx = (didx0, didx1)
    rows = (rows0, rows1, rows2, rows3)
    gsem = (gsem0, gsem1, gsem2, gsem3)
    ssem = (ssem0, ssem1, ssem2, ssem3)
    isem = (isem0, isem1)

    c = lax.axis_index("c")
    s = lax.axis_index("s")
    wid = s * NC + c
    wbase = wid * WPW

    z16 = jnp.zeros((16,), jnp.float32)
    iota16 = lax.iota(jnp.int32, 16)
    for j in range(WIN // 16):
        pidx[pl.ds(j * 16, 16)] = iota16 + (
            N_NODES + (s * WIN + j * 16) % (N_PAD - N_NODES - 16))
        pidx_lo[pl.ds(j * 16, 16)] = iota16 + s * WIN + j * 16

    def _zrow(i, carry):
        for j in range(D // 16):
            rows0[i, pl.ds(j * 16, 16)] = z16
        return carry

    lax.fori_loop(0, WIN, _zrow, 0)

    if compute_cnt:
        one16 = jnp.ones((16,), jnp.float32)
        for j in range(WIN // 16):
            ones_v[pl.ds(j * 16, 16)] = one16

    base = s * RPT
    for t in range(RPT // WIN):
        pltpu.sync_copy(rows0, agg_sh.at[pl.ds(base + t * WIN, WIN)])
    if compute_cnt:
        for t in range(RPT // D):
            pltpu.sync_copy(rows0.at[0], cnt_sh.at[pl.ds(base + t * D, D)])
    plsc.subcore_barrier()

    def wait_gather(b):
        pltpu.make_async_copy(x_hbm.at[pidx_lo], rows[b], gsem[b]).wait()

    def wait_scatter(b):
        pltpu.make_async_copy(rows[b], agg_sh.at[pidx], ssem[b]).wait()
        if compute_cnt:
            pltpu.make_async_copy(ones_v, cnt_sh.at[pidx], csem[b]).wait()

    def wait_idx(gb, off):
        pltpu.make_async_copy(src_hbm.at[pl.ds(off, G)], sidx[gb],
                              isem[gb]).wait()
        pltpu.make_async_copy(dst_hbm.at[pl.ds(off, G)], didx[gb],
                              isem[gb]).wait()

    pltpu.async_copy(src_hbm.at[pl.ds(wbase, G)], sidx0, isem0)
    pltpu.async_copy(dst_hbm.at[pl.ds(wbase, G)], didx0, isem0)
    for j in range(WIN // 16):
        didx1[G - 2, pl.ds(j * 16, 16)] = pidx[pl.ds(j * 16, 16)]
        didx1[G - 1, pl.ds(j * 16, 16)] = pidx[pl.ds(j * 16, 16)]
    pltpu.async_copy(x_hbm.at[pidx_lo], rows2, gsem2)
    pltpu.async_copy(x_hbm.at[pidx_lo], rows3, gsem3)
    for b in range(NBUF):
        pltpu.async_copy(rows[b], agg_sh.at[pidx], ssem[b], add=True)
        if compute_cnt:
            pltpu.async_copy(ones_v, cnt_sh.at[pidx], csem[b], add=True)

    def pair_body(gg, carry):
        for gpar in range(2):
            gb = gpar
            g = 2 * gg + gpar
            gwbase = wbase + g * G
            wait_idx(gb, gwbase)
            for k in range(G):
                b = k % NBUF
                bv = (k + LAG) % NBUF
                wait_scatter(b)
                pltpu.async_copy(x_hbm.at[sidx[gb].at[k]], rows[b], gsem[b])
                wait_gather(bv)
                if k < LAG:
                    dv = didx[gb ^ 1].at[G - LAG + k]
                else:
                    dv = didx[gb].at[k - LAG]
                pltpu.async_copy(rows[bv], agg_sh.at[dv], ssem[bv], add=True)
                if compute_cnt:
                    pltpu.async_copy(ones_v, cnt_sh.at[dv], csem[bv],
                                     add=True)
                if k == 3:
                    nxt = wbase + (g + 1) * G
                    pltpu.async_copy(src_hbm.at[pl.ds(nxt, G)], sidx[gb ^ 1],
                                     isem[gb ^ 1])
                    pltpu.async_copy(dst_hbm.at[pl.ds(nxt, G)], didx[gb ^ 1],
                                     isem[gb ^ 1])
        return carry

    lax.fori_loop(0, NG // 2, pair_body, 0)

    for e in range(LAG):
        v = WPW - LAG + e
        kv = G - LAG + e
        bv = v % NBUF
        wait_gather(bv)
        pltpu.async_copy(rows[bv], agg_sh.at[didx1.at[kv]], ssem[bv],
                         add=True)
        if compute_cnt:
            pltpu.async_copy(ones_v, cnt_sh.at[didx1.at[kv]], csem[bv],
                             add=True)
    for b in range(NBUF):
        wait_scatter(b)
    wait_scatter(2)
    wait_scatter(3)
    wait_idx(0, wbase + NG * G)

    plsc.subcore_barrier()

    for t in range(RPT // 128):
        pltpu.sync_copy(agg_sh.at[pl.ds(base + t * 128, 128)],
                        agg_out.at[c, pl.ds(base + t * 128, 128)])
    if compute_cnt:
        pltpu.sync_copy(cnt_sh.at[pl.ds(base, RPT)],
                        cnt_out.at[c, pl.ds(base, RPT)])


def _make_sc_agg(compute_cnt):
    mesh = plsc.VectorSubcoreMesh(core_axis_name="c", subcore_axis_name="s",
                                  num_cores=NC, num_subcores=NS)
    out_type = [jax.ShapeDtypeStruct((NC, N_PAD, D), jnp.float32)]
    if compute_cnt:
        out_type.append(jax.ShapeDtypeStruct((NC, N_PAD), jnp.float32))
    scratch = [
        pltpu.VMEM((G, WIN), jnp.int32),
        pltpu.VMEM((G, WIN), jnp.int32),
        pltpu.VMEM((G, WIN), jnp.int32),
        pltpu.VMEM((G, WIN), jnp.int32),
        pltpu.VMEM((WIN, D), jnp.float32),
        pltpu.VMEM((WIN, D), jnp.float32),
        pltpu.VMEM((WIN, D), jnp.float32),
        pltpu.VMEM((WIN, D), jnp.float32),
        pltpu.VMEM((WIN,), jnp.int32),
        pltpu.VMEM((WIN,), jnp.int32),
    ]
    if compute_cnt:
        scratch.append(pltpu.VMEM((WIN,), jnp.float32))
    scratch.append(pltpu.VMEM_SHARED((N_PAD, D), jnp.float32))
    if compute_cnt:
        scratch.append(pltpu.VMEM_SHARED((N_PAD,), jnp.float32))
    nsem = 10 + (4 if compute_cnt else 0)
    scratch += [pltpu.SemaphoreType.DMA] * nsem

    return pl.kernel(
        functools.partial(_sc_agg_body, compute_cnt),
        out_type=tuple(out_type),
        mesh=mesh,
        scratch_types=scratch,
        name=f"sage_sc_agg_cnt{int(compute_cnt)}",
    )


_SC_AGG_CNT = _make_sc_agg(True)
_SC_AGG = _make_sc_agg(False)


def _combine_body(relu, a0, a1, c0, c1, xr, wn, ws, br, o):
    cnt = c0[...] + c1[...]
    r = 1.0 / jnp.maximum(cnt, 1.0)
    mean = (a0[...] + a1[...]) * r
    acc = jnp.dot(mean, wn[...], preferred_element_type=jnp.float32)
    acc = acc + jnp.dot(xr[...], ws[...], preferred_element_type=jnp.float32)
    acc = acc + br[...]
    if relu:
        acc = jnp.maximum(acc, 0.0)
    o[...] = acc


_R_BLK = 2000


def _combine(a0, a1, c0, c1, xr, wn, ws, br, relu):
    grid = (N_NODES // _R_BLK,)
    row_spec = pl.BlockSpec((_R_BLK, D), lambda i: (i, 0))
    cnt_spec = pl.BlockSpec((_R_BLK, 1), lambda i: (i, 0))
    w_spec = pl.BlockSpec((D, D), lambda i: (0, 0))
    b_spec = pl.BlockSpec((1, D), lambda i: (0, 0))
    return pl.pallas_call(
        functools.partial(_combine_body, relu),
        grid=grid,
        in_specs=[row_spec, row_spec, cnt_spec, cnt_spec, row_spec,
                  w_spec, w_spec, b_spec],
        out_specs=row_spec,
        out_shape=jax.ShapeDtypeStruct((N_NODES, D), jnp.float32),
        name=f"sage_combine_relu{int(relu)}",
    )(a0, a1, c0, c1, xr, wn, ws, br)


def kernel(x, edge_index, W1_self, W1_neigh, b1, W2_self, W2_neigh, b2):
    src = edge_index[0]
    dst = edge_index[1]
    pad = E_PAD - N_EDGES
    pad_ids = jnp.arange(pad, dtype=jnp.int32)
    src_p = jnp.concatenate([src, pad_ids % N_NODES]).reshape(IDX_WINDOWS, WIN)
    dst_p = jnp.concatenate([dst, N_NODES + pad_ids % (N_PAD - N_NODES)]
                            ).reshape(IDX_WINDOWS, WIN)

    agg1, cnt = _SC_AGG_CNT(x, src_p, dst_p)
    c0 = cnt[0].reshape(N_PAD, 1)
    c1 = cnt[1].reshape(N_PAD, 1)
    b1r = b1.reshape(1, D)
    b2r = b2.reshape(1, D)

    h = _combine(agg1[0], agg1[1], c0, c1, x, W1_neigh, W1_self, b1r,
                 relu=True)
    (agg2,) = _SC_AGG(h, src_p, dst_p)
    out = _combine(agg2[0], agg2[1], c0, c1, h, W2_neigh, W2_self, b2r,
                   relu=False)
    return out


# speedup vs baseline: 13.7831x; 1.0142x over previous
import functools

import jax
import jax.numpy as jnp
from jax import lax
from jax.experimental import pallas as pl
from jax.experimental.pallas import tpu as pltpu
from jax.experimental.pallas import tpu_sc as plsc

N_NODES = 10000
N_EDGES = 320000
D = 128

NC = 2
NS = 16
NW = NC * NS

WIN = 64
G = 8
N_WINDOWS = 5120
WPW = N_WINDOWS // NW
NG = WPW // G
IDX_WINDOWS = N_WINDOWS + G
E_PAD = IDX_WINDOWS * WIN
N_PAD = 10240
RPT = N_PAD // NS
NBUF = 4
LAG = 2


def _sc_agg_body(compute_cnt, x_hbm, src_hbm, dst_hbm, *refs):
    if compute_cnt:
        (agg_out, cnt_out, sidx0, sidx1, didx0, didx1,
         rows0, rows1, rows2, rows3, pidx, pidx_lo, ones_v,
         agg_sh, cnt_sh,
         gsem0, gsem1, gsem2, gsem3, ssem0, ssem1, ssem2, ssem3,
         csem0, csem1, csem2, csem3, isem0, isem1) = refs
        csem = (csem0, csem1, csem2, csem3)
    else:
        (agg_out, sidx0, sidx1, didx0, didx1,
         rows0, rows1, rows2, rows3, pidx, pidx_lo,
         agg_sh,
         gsem0, gsem1, gsem2, gsem3, ssem0, ssem1, ssem2, ssem3,
         isem0, isem1) = refs

    sidx = (sidx0, sidx1)
    didx = (didx0, didx1)
    rows = (rows0, rows1, rows2, rows3)
    gsem = (gsem0, gsem1, gsem2, gsem3)
    ssem = (ssem0, ssem1, ssem2, ssem3)
    isem = (isem0, isem1)

    c = lax.axis_index("c")
    s = lax.axis_index("s")
    wid = s * NC + c
    wbase = wid * WPW

    z16 = jnp.zeros((16,), jnp.float32)
    iota16 = lax.iota(jnp.int32, 16)
    for j in range(WIN // 16):
        pidx[pl.ds(j * 16, 16)] = iota16 + (
            N_NODES + (s * WIN + j * 16) % (N_PAD - N_NODES - 16))
        pidx_lo[pl.ds(j * 16, 16)] = iota16 + s * WIN + j * 16

    def _zrow(i, carry):
        for j in range(D // 16):
            rows0[i, pl.ds(j * 16, 16)] = z16
        return carry

    lax.fori_loop(0, WIN, _zrow, 0)

    if compute_cnt:
        one16 = jnp.ones((16,), jnp.float32)
        for j in range(WIN // 16):
            ones_v[pl.ds(j * 16, 16)] = one16

    base = s * RPT
    for t in range(RPT // WIN):
        pltpu.sync_copy(rows0, agg_sh.at[pl.ds(base + t * WIN, WIN)])
    if compute_cnt:
        for t in range(RPT // D):
            pltpu.sync_copy(rows0.at[0], cnt_sh.at[pl.ds(base + t * D, D)])
    plsc.subcore_barrier()

    def wait_gather(b):
        pltpu.make_async_copy(x_hbm.at[pidx_lo], rows[b], gsem[b]).wait()

    def wait_scatter(b):
        pltpu.make_async_copy(rows[b], agg_sh.at[pidx], ssem[b]).wait()
        if compute_cnt:
            pltpu.make_async_copy(ones_v, cnt_sh.at[pidx], csem[b]).wait()

    def wait_idx(gb, off):
        pltpu.make_async_copy(src_hbm.at[pl.ds(off, G)], sidx[gb],
                              isem[gb]).wait()
        pltpu.make_async_copy(dst_hbm.at[pl.ds(off, G)], didx[gb],
                              isem[gb]).wait()

    pltpu.async_copy(src_hbm.at[pl.ds(wbase, G)], sidx0, isem0)
    pltpu.async_copy(dst_hbm.at[pl.ds(wbase, G)], didx0, isem0)
    for j in range(WIN // 16):
        didx1[G - 2, pl.ds(j * 16, 16)] = pidx[pl.ds(j * 16, 16)]
        didx1[G - 1, pl.ds(j * 16, 16)] = pidx[pl.ds(j * 16, 16)]
    pltpu.async_copy(x_hbm.at[pidx_lo], rows2, gsem2)
    pltpu.async_copy(x_hbm.at[pidx_lo], rows3, gsem3)
    for b in range(LAG):
        pltpu.async_copy(rows[b], agg_sh.at[pidx], ssem[b], add=True)
        if compute_cnt:
            pltpu.async_copy(ones_v, cnt_sh.at[pidx], csem[b], add=True)

    def pair_body(gg, carry):
        for gpar in range(2):
            gb = gpar
            g = 2 * gg + gpar
            gwbase = wbase + g * G
            wait_idx(gb, gwbase)
            for k in range(G):
                b = k % NBUF
                bv = (k + LAG) % NBUF
                wait_scatter(b)
                pltpu.async_copy(x_hbm.at[sidx[gb].at[k]], rows[b], gsem[b])
                wait_gather(bv)
                if k < LAG:
                    dv = didx[gb ^ 1].at[G - LAG + k]
                else:
                    dv = didx[gb].at[k - LAG]
                pltpu.async_copy(rows[bv], agg_sh.at[dv], ssem[bv], add=True)
                if compute_cnt:
                    pltpu.async_copy(ones_v, cnt_sh.at[dv], csem[bv],
                                     add=True)
                if k == 3:
                    nxt = wbase + (g + 1) * G
                    pltpu.async_copy(src_hbm.at[pl.ds(nxt, G)], sidx[gb ^ 1],
                                     isem[gb ^ 1])
                    pltpu.async_copy(dst_hbm.at[pl.ds(nxt, G)], didx[gb ^ 1],
                                     isem[gb ^ 1])
        return carry

    lax.fori_loop(0, NG // 2, pair_body, 0)

    for e in range(LAG):
        v = WPW - LAG + e
        kv = G - LAG + e
        bv = v % NBUF
        wait_gather(bv)
        pltpu.async_copy(rows[bv], agg_sh.at[didx1.at[kv]], ssem[bv],
                         add=True)
        if compute_cnt:
            pltpu.async_copy(ones_v, cnt_sh.at[didx1.at[kv]], csem[bv],
                             add=True)
    for b in range(NBUF):
        wait_scatter(b)
    wait_idx(0, wbase + NG * G)

    plsc.subcore_barrier()

    for t in range(RPT // 128):
        pltpu.sync_copy(agg_sh.at[pl.ds(base + t * 128, 128)],
                        agg_out.at[c, pl.ds(base + t * 128, 128)])
    if compute_cnt:
        pltpu.sync_copy(cnt_sh.at[pl.ds(base, RPT)],
                        cnt_out.at[c, pl.ds(base, RPT)])


def _make_sc_agg(compute_cnt):
    mesh = plsc.VectorSubcoreMesh(core_axis_name="c", subcore_axis_name="s",
                                  num_cores=NC, num_subcores=NS)
    out_type = [jax.ShapeDtypeStruct((NC, N_PAD, D), jnp.float32)]
    if compute_cnt:
        out_type.append(jax.ShapeDtypeStruct((NC, N_PAD), jnp.float32))
    scratch = [
        pltpu.VMEM((G, WIN), jnp.int32),
        pltpu.VMEM((G, WIN), jnp.int32),
        pltpu.VMEM((G, WIN), jnp.int32),
        pltpu.VMEM((G, WIN), jnp.int32),
        pltpu.VMEM((WIN, D), jnp.float32),
        pltpu.VMEM((WIN, D), jnp.float32),
        pltpu.VMEM((WIN, D), jnp.float32),
        pltpu.VMEM((WIN, D), jnp.float32),
        pltpu.VMEM((WIN,), jnp.int32),
        pltpu.VMEM((WIN,), jnp.int32),
    ]
    if compute_cnt:
        scratch.append(pltpu.VMEM((WIN,), jnp.float32))
    scratch.append(pltpu.VMEM_SHARED((N_PAD, D), jnp.float32))
    if compute_cnt:
        scratch.append(pltpu.VMEM_SHARED((N_PAD,), jnp.float32))
    nsem = 10 + (4 if compute_cnt else 0)
    scratch += [pltpu.SemaphoreType.DMA] * nsem

    return pl.kernel(
        functools.partial(_sc_agg_body, compute_cnt),
        out_type=tuple(out_type),
        mesh=mesh,
        scratch_types=scratch,
        name=f"sage_sc_agg_cnt{int(compute_cnt)}",
    )


_SC_AGG_CNT = _make_sc_agg(True)
_SC_AGG = _make_sc_agg(False)


def _combine_body(relu, a0, a1, c0, c1, xr, wn, ws, br, o):
    cnt = c0[...] + c1[...]
    r = 1.0 / jnp.maximum(cnt, 1.0)
    mean = (a0[...] + a1[...]) * r
    acc = jnp.dot(mean, wn[...], preferred_element_type=jnp.float32)
    acc = acc + jnp.dot(xr[...], ws[...], preferred_element_type=jnp.float32)
    acc = acc + br[...]
    if relu:
        acc = jnp.maximum(acc, 0.0)
    o[...] = acc


_R_BLK = 2000


def _combine(a0, a1, c0, c1, xr, wn, ws, br, relu):
    grid = (N_NODES // _R_BLK,)
    row_spec = pl.BlockSpec((_R_BLK, D), lambda i: (i, 0))
    cnt_spec = pl.BlockSpec((_R_BLK, 1), lambda i: (i, 0))
    w_spec = pl.BlockSpec((D, D), lambda i: (0, 0))
    b_spec = pl.BlockSpec((1, D), lambda i: (0, 0))
    return pl.pallas_call(
        functools.partial(_combine_body, relu),
        grid=grid,
        in_specs=[row_spec, row_spec, cnt_spec, cnt_spec, row_spec,
                  w_spec, w_spec, b_spec],
        out_specs=row_spec,
        out_shape=jax.ShapeDtypeStruct((N_NODES, D), jnp.float32),
        name=f"sage_combine_relu{int(relu)}",
    )(a0, a1, c0, c1, xr, wn, ws, br)


def kernel(x, edge_index, W1_self, W1_neigh, b1, W2_self, W2_neigh, b2):
    src = edge_index[0]
    dst = edge_index[1]
    pad = E_PAD - N_EDGES
    pad_ids = jnp.arange(pad, dtype=jnp.int32)
    src_p = jnp.concatenate([src, pad_ids % N_NODES]).reshape(IDX_WINDOWS, WIN)
    dst_p = jnp.concatenate([dst, N_NODES + pad_ids % (N_PAD - N_NODES)]
                            ).reshape(IDX_WINDOWS, WIN)

    agg1, cnt = _SC_AGG_CNT(x, src_p, dst_p)
    c0 = cnt[0].reshape(N_PAD, 1)
    c1 = cnt[1].reshape(N_PAD, 1)
    b1r = b1.reshape(1, D)
    b2r = b2.reshape(1, D)

    h = _combine(agg1[0], agg1[1], c0, c1, x, W1_neigh, W1_self, b1r,
                 relu=True)
    (agg2,) = _SC_AGG(h, src_p, dst_p)
    out = _combine(agg2[0], agg2[1], c0, c1, h, W2_neigh, W2_self, b2r,
                   relu=False)
    return out


# speedup vs baseline: 14.5049x; 1.0524x over previous
import functools

import jax
import jax.numpy as jnp
import numpy as np
from jax import lax
from jax.experimental import pallas as pl
from jax.experimental.pallas import tpu as pltpu
from jax.experimental.pallas import tpu_sc as plsc

N_NODES = 10000
N_EDGES = 320000
D = 128

NC = 2
NS = 16
NW = NC * NS

WIN = 64
G = 8
N_WINDOWS = 5120
WPW = N_WINDOWS // NW
NG = WPW // G
IDX_WINDOWS = N_WINDOWS + G
E_PAD = IDX_WINDOWS * WIN
N_PAD = 10240
RPT = N_PAD // NS
NBUF = 4
LAG = 2


def _sc_agg_body(compute_cnt, x_hbm, src_hbm, dst_hbm, *refs):
    if compute_cnt:
        (agg_out, cnt_out, sidx0, sidx1, didx0, didx1,
         rows0, rows1, rows2, rows3, pidx, pidx_lo, ones_v,
         agg_sh, cnt_sh,
         gsem0, gsem1, gsem2, gsem3, ssem0, ssem1, ssem2, ssem3,
         csem0, csem1, csem2, csem3, isem0, isem1) = refs
        csem = (csem0, csem1, csem2, csem3)
    else:
        (agg_out, sidx0, sidx1, didx0, didx1,
         rows0, rows1, rows2, rows3, pidx, pidx_lo,
         agg_sh,
         gsem0, gsem1, gsem2, gsem3, ssem0, ssem1, ssem2, ssem3,
         isem0, isem1) = refs

    sidx = (sidx0, sidx1)
    didx = (didx0, didx1)
    rows = (rows0, rows1, rows2, rows3)
    gsem = (gsem0, gsem1, gsem2, gsem3)
    ssem = (ssem0, ssem1, ssem2, ssem3)
    isem = (isem0, isem1)

    c = lax.axis_index("c")
    s = lax.axis_index("s")
    wid = s * NC + c
    wbase = wid * WPW

    z16 = jnp.zeros((16,), jnp.float32)
    iota16 = lax.iota(jnp.int32, 16)
    for j in range(WIN // 16):
        pidx[pl.ds(j * 16, 16)] = iota16 + (
            N_NODES + (s * WIN + j * 16) % (N_PAD - N_NODES - 16))
        pidx_lo[pl.ds(j * 16, 16)] = iota16 + s * WIN + j * 16

    def _zrow(i, carry):
        for j in range(D // 16):
            rows0[i, pl.ds(j * 16, 16)] = z16
        return carry

    lax.fori_loop(0, WIN, _zrow, 0)

    if compute_cnt:
        one16 = jnp.ones((16,), jnp.float32)
        for j in range(WIN // 16):
            ones_v[pl.ds(j * 16, 16)] = one16

    base = s * RPT
    for t in range(RPT // WIN):
        pltpu.sync_copy(rows0, agg_sh.at[pl.ds(base + t * WIN, WIN)])
    if compute_cnt:
        for t in range(RPT // D):
            pltpu.sync_copy(rows0.at[0], cnt_sh.at[pl.ds(base + t * D, D)])
    plsc.subcore_barrier()

    def wait_gather(b):
        pltpu.make_async_copy(x_hbm.at[pidx_lo], rows[b], gsem[b]).wait()

    def wait_scatter(b):
        pltpu.make_async_copy(rows[b], agg_sh.at[pidx], ssem[b]).wait()
        if compute_cnt:
            pltpu.make_async_copy(ones_v, cnt_sh.at[pidx], csem[b]).wait()

    def wait_idx(gb, off):
        pltpu.make_async_copy(src_hbm.at[pl.ds(off, G)], sidx[gb],
                              isem[gb]).wait()
        pltpu.make_async_copy(dst_hbm.at[pl.ds(off, G)], didx[gb],
                              isem[gb]).wait()

    pltpu.async_copy(src_hbm.at[pl.ds(wbase, G)], sidx0, isem0)
    pltpu.async_copy(dst_hbm.at[pl.ds(wbase, G)], didx0, isem0)
    for j in range(WIN // 16):
        didx1[G - 2, pl.ds(j * 16, 16)] = pidx[pl.ds(j * 16, 16)]
        didx1[G - 1, pl.ds(j * 16, 16)] = pidx[pl.ds(j * 16, 16)]
    pltpu.async_copy(x_hbm.at[pidx_lo], rows2, gsem2)
    pltpu.async_copy(x_hbm.at[pidx_lo], rows3, gsem3)
    for b in range(LAG):
        pltpu.async_copy(rows[b], agg_sh.at[pidx], ssem[b], add=True)
        if compute_cnt:
            pltpu.async_copy(ones_v, cnt_sh.at[pidx], csem[b], add=True)

    def pair_body(gg, carry):
        for gpar in range(2):
            gb = gpar
            g = 2 * gg + gpar
            gwbase = wbase + g * G
            wait_idx(gb, gwbase)
            for k in range(G):
                b = k % NBUF
                bv = (k + LAG) % NBUF
                wait_scatter(b)
                pltpu.async_copy(x_hbm.at[sidx[gb].at[k]], rows[b], gsem[b])
                wait_gather(bv)
                if k < LAG:
                    dv = didx[gb ^ 1].at[G - LAG + k]
                else:
                    dv = didx[gb].at[k - LAG]
                pltpu.async_copy(rows[bv], agg_sh.at[dv], ssem[bv], add=True)
                if compute_cnt:
                    pltpu.async_copy(ones_v, cnt_sh.at[dv], csem[bv],
                                     add=True)
                if k == 3:
                    nxt = wbase + (g + 1) * G
                    pltpu.async_copy(src_hbm.at[pl.ds(nxt, G)], sidx[gb ^ 1],
                                     isem[gb ^ 1])
                    pltpu.async_copy(dst_hbm.at[pl.ds(nxt, G)], didx[gb ^ 1],
                                     isem[gb ^ 1])
        return carry

    lax.fori_loop(0, NG // 2, pair_body, 0)

    for e in range(LAG):
        v = WPW - LAG + e
        kv = G - LAG + e
        bv = v % NBUF
        wait_gather(bv)
        pltpu.async_copy(rows[bv], agg_sh.at[didx1.at[kv]], ssem[bv],
                         add=True)
        if compute_cnt:
            pltpu.async_copy(ones_v, cnt_sh.at[didx1.at[kv]], csem[bv],
                             add=True)
    for b in range(NBUF):
        wait_scatter(b)
    wait_idx(0, wbase + NG * G)

    plsc.subcore_barrier()

    for t in range(RPT // 128):
        pltpu.sync_copy(agg_sh.at[pl.ds(base + t * 128, 128)],
                        agg_out.at[c, pl.ds(base + t * 128, 128)])
    if compute_cnt:
        pltpu.sync_copy(cnt_sh.at[pl.ds(base, RPT)],
                        cnt_out.at[c, pl.ds(base, RPT)])


def _make_sc_agg(compute_cnt):
    mesh = plsc.VectorSubcoreMesh(core_axis_name="c", subcore_axis_name="s",
                                  num_cores=NC, num_subcores=NS)
    out_type = [jax.ShapeDtypeStruct((NC, N_PAD, D), jnp.float32)]
    if compute_cnt:
        out_type.append(jax.ShapeDtypeStruct((NC, N_PAD), jnp.float32))
    scratch = [
        pltpu.VMEM((G, WIN), jnp.int32),
        pltpu.VMEM((G, WIN), jnp.int32),
        pltpu.VMEM((G, WIN), jnp.int32),
        pltpu.VMEM((G, WIN), jnp.int32),
        pltpu.VMEM((WIN, D), jnp.float32),
        pltpu.VMEM((WIN, D), jnp.float32),
        pltpu.VMEM((WIN, D), jnp.float32),
        pltpu.VMEM((WIN, D), jnp.float32),
        pltpu.VMEM((WIN,), jnp.int32),
        pltpu.VMEM((WIN,), jnp.int32),
    ]
    if compute_cnt:
        scratch.append(pltpu.VMEM((WIN,), jnp.float32))
    scratch.append(pltpu.VMEM_SHARED((N_PAD, D), jnp.float32))
    if compute_cnt:
        scratch.append(pltpu.VMEM_SHARED((N_PAD,), jnp.float32))
    nsem = 10 + (4 if compute_cnt else 0)
    scratch += [pltpu.SemaphoreType.DMA] * nsem

    return pl.kernel(
        functools.partial(_sc_agg_body, compute_cnt),
        out_type=tuple(out_type),
        mesh=mesh,
        scratch_types=scratch,
        name=f"sage_sc_agg_cnt{int(compute_cnt)}",
    )


_SC_AGG_CNT = _make_sc_agg(True)
_SC_AGG = _make_sc_agg(False)


def _combine_body(relu, a0, a1, c0, c1, xr, wn, ws, br, o):
    cnt = c0[0] + c1[0]
    r = 1.0 / jnp.maximum(cnt, 1.0)
    mean = (a0[0] + a1[0]) * r
    acc = jnp.dot(mean, wn[...], preferred_element_type=jnp.float32)
    acc = acc + jnp.dot(xr[...], ws[...], preferred_element_type=jnp.float32)
    acc = acc + br[...]
    if relu:
        acc = jnp.maximum(acc, 0.0)
    o[...] = acc


_R_BLK = 2000


def _combine(agg, cnt3, xr, wn, ws, br, relu):
    grid = (N_NODES // _R_BLK,)
    row_spec = pl.BlockSpec((_R_BLK, D), lambda i: (i, 0))
    a0_spec = pl.BlockSpec((1, _R_BLK, D), lambda i: (0, i, 0))
    a1_spec = pl.BlockSpec((1, _R_BLK, D), lambda i: (1, i, 0))
    c0_spec = pl.BlockSpec((1, _R_BLK, 1), lambda i: (0, i, 0))
    c1_spec = pl.BlockSpec((1, _R_BLK, 1), lambda i: (1, i, 0))
    w_spec = pl.BlockSpec((D, D), lambda i: (0, 0))
    b_spec = pl.BlockSpec((1, D), lambda i: (0, 0))
    return pl.pallas_call(
        functools.partial(_combine_body, relu),
        grid=grid,
        in_specs=[a0_spec, a1_spec, c0_spec, c1_spec, row_spec,
                  w_spec, w_spec, b_spec],
        out_specs=row_spec,
        out_shape=jax.ShapeDtypeStruct((N_NODES, D), jnp.float32),
        name=f"sage_combine_relu{int(relu)}",
    )(agg, agg, cnt3, cnt3, xr, wn, ws, br)


_PAD_N = E_PAD - N_EDGES
_PAD_SRC = jnp.asarray(np.arange(_PAD_N, dtype=np.int32) % N_NODES)
_PAD_DST = jnp.asarray(
    N_NODES + np.arange(_PAD_N, dtype=np.int32) % (N_PAD - N_NODES))


def kernel(x, edge_index, W1_self, W1_neigh, b1, W2_self, W2_neigh, b2):
    src = edge_index[0]
    dst = edge_index[1]
    src_p = jnp.concatenate([src, _PAD_SRC]).reshape(IDX_WINDOWS, WIN)
    dst_p = jnp.concatenate([dst, _PAD_DST]).reshape(IDX_WINDOWS, WIN)

    agg1, cnt = _SC_AGG_CNT(x, src_p, dst_p)
    cnt3 = cnt.reshape(NC, N_PAD, 1)
    b1r = b1.reshape(1, D)
    b2r = b2.reshape(1, D)

    h = _combine(agg1, cnt3, x, W1_neigh, W1_self, b1r, relu=True)
    (agg2,) = _SC_AGG(h, src_p, dst_p)
    out = _combine(agg2, cnt3, h, W2_neigh, W2_self, b2r, relu=False)
    return out


# speedup vs baseline: 14.5075x; 1.0002x over previous
import functools

import jax
import jax.numpy as jnp
import numpy as np
from jax import lax
from jax.experimental import pallas as pl
from jax.experimental.pallas import tpu as pltpu
from jax.experimental.pallas import tpu_sc as plsc

N_NODES = 10000
N_EDGES = 320000
D = 128

NC = 2
NS = 16
NW = NC * NS

WIN = 64
G = 8
N_WINDOWS = 5120
WPW = N_WINDOWS // NW
NG = WPW // G
IDX_WINDOWS = N_WINDOWS + G
E_PAD = IDX_WINDOWS * WIN
N_PAD = 10240
RPT = N_PAD // NS
NBUF = 4
LAG = 2


def _sc_agg_body(compute_cnt, x_hbm, src_hbm, dst_hbm, *refs):
    if compute_cnt:
        (agg_out, cnt_out, sidx0, sidx1, didx0, didx1,
         rows0, rows1, rows2, rows3, pidx, pidx_lo, ones_v,
         agg_sh, cnt_sh,
         gsem0, gsem1, gsem2, gsem3, ssem0, ssem1, ssem2, ssem3,
         csem0, csem1, csem2, csem3, isem0, isem1) = refs
        csem = (csem0, csem1, csem2, csem3)
    else:
        (agg_out, sidx0, sidx1, didx0, didx1,
         rows0, rows1, rows2, rows3, pidx, pidx_lo,
         agg_sh,
         gsem0, gsem1, gsem2, gsem3, ssem0, ssem1, ssem2, ssem3,
         isem0, isem1) = refs

    sidx = (sidx0, sidx1)
    didx = (didx0, didx1)
    rows = (rows0, rows1, rows2, rows3)
    gsem = (gsem0, gsem1, gsem2, gsem3)
    ssem = (ssem0, ssem1, ssem2, ssem3)
    isem = (isem0, isem1)

    c = lax.axis_index("c")
    s = lax.axis_index("s")
    wid = s * NC + c
    wbase = wid * WPW

    z16 = jnp.zeros((16,), jnp.float32)
    iota16 = lax.iota(jnp.int32, 16)
    for j in range(WIN // 16):
        pidx[pl.ds(j * 16, 16)] = iota16 + (
            N_NODES + (s * WIN + j * 16) % (N_PAD - N_NODES - 16))
        pidx_lo[pl.ds(j * 16, 16)] = iota16 + s * WIN + j * 16

    def _zrow(i, carry):
        for j in range(D // 16):
            rows0[i, pl.ds(j * 16, 16)] = z16
        return carry

    lax.fori_loop(0, WIN, _zrow, 0)

    if compute_cnt:
        one16 = jnp.ones((16,), jnp.float32)
        for j in range(WIN // 16):
            ones_v[pl.ds(j * 16, 16)] = one16

    base = s * RPT
    for t in range(RPT // WIN):
        pltpu.sync_copy(rows0, agg_sh.at[pl.ds(base + t * WIN, WIN)])
    if compute_cnt:
        for t in range(RPT // D):
            pltpu.sync_copy(rows0.at[0], cnt_sh.at[pl.ds(base + t * D, D)])
    plsc.subcore_barrier()

    def wait_gather(b):
        pltpu.make_async_copy(x_hbm.at[pidx_lo], rows[b], gsem[b]).wait()

    def wait_scatter(b):
        pltpu.make_async_copy(rows[b], agg_sh.at[pidx], ssem[b]).wait()
        if compute_cnt:
            pltpu.make_async_copy(ones_v, cnt_sh.at[pidx], csem[b]).wait()

    def wait_idx(gb, off):
        pltpu.make_async_copy(src_hbm.at[pl.ds(off, G)], sidx[gb],
                              isem[gb]).wait()
        pltpu.make_async_copy(dst_hbm.at[pl.ds(off, G)], didx[gb],
                              isem[gb]).wait()

    pltpu.async_copy(src_hbm.at[pl.ds(wbase, G)], sidx0, isem0)
    pltpu.async_copy(dst_hbm.at[pl.ds(wbase, G)], didx0, isem0)
    for j in range(WIN // 16):
        didx1[G - 2, pl.ds(j * 16, 16)] = pidx[pl.ds(j * 16, 16)]
        didx1[G - 1, pl.ds(j * 16, 16)] = pidx[pl.ds(j * 16, 16)]
    pltpu.async_copy(x_hbm.at[pidx_lo], rows2, gsem2)
    pltpu.async_copy(x_hbm.at[pidx_lo], rows3, gsem3)
    for b in range(LAG):
        pltpu.async_copy(rows[b], agg_sh.at[pidx], ssem[b], add=True)
        if compute_cnt:
            pltpu.async_copy(ones_v, cnt_sh.at[pidx], csem[b], add=True)

    def pair_body(gg, carry):
        for gpar in range(2):
            gb = gpar
            g = 2 * gg + gpar
            gwbase = wbase + g * G
            wait_idx(gb, gwbase)
            for k in range(G):
                b = k % NBUF
                bv = (k + LAG) % NBUF
                wait_scatter(b)
                pltpu.async_copy(x_hbm.at[sidx[gb].at[k]], rows[b], gsem[b])
                wait_gather(bv)
                if k < LAG:
                    dv = didx[gb ^ 1].at[G - LAG + k]
                else:
                    dv = didx[gb].at[k - LAG]
                pltpu.async_copy(rows[bv], agg_sh.at[dv], ssem[bv], add=True)
                if compute_cnt:
                    pltpu.async_copy(ones_v, cnt_sh.at[dv], csem[bv],
                                     add=True)
                if k == 3:
                    nxt = wbase + (g + 1) * G
                    pltpu.async_copy(src_hbm.at[pl.ds(nxt, G)], sidx[gb ^ 1],
                                     isem[gb ^ 1])
                    pltpu.async_copy(dst_hbm.at[pl.ds(nxt, G)], didx[gb ^ 1],
                                     isem[gb ^ 1])
        return carry

    lax.fori_loop(0, NG // 2, pair_body, 0)

    for e in range(LAG):
        v = WPW - LAG + e
        kv = G - LAG + e
        bv = v % NBUF
        wait_gather(bv)
        pltpu.async_copy(rows[bv], agg_sh.at[didx1.at[kv]], ssem[bv],
                         add=True)
        if compute_cnt:
            pltpu.async_copy(ones_v, cnt_sh.at[didx1.at[kv]], csem[bv],
                             add=True)
    for b in range(NBUF):
        wait_scatter(b)
    wait_idx(0, wbase + NG * G)

    plsc.subcore_barrier()

    for t in range(RPT // 128):
        pltpu.sync_copy(agg_sh.at[pl.ds(base + t * 128, 128)],
                        agg_out.at[c, pl.ds(base + t * 128, 128)])
    if compute_cnt:
        pltpu.sync_copy(cnt_sh.at[pl.ds(base, RPT)],
                        cnt_out.at[c, pl.ds(base, RPT)])


def _make_sc_agg(compute_cnt):
    mesh = plsc.VectorSubcoreMesh(core_axis_name="c", subcore_axis_name="s",
                                  num_cores=NC, num_subcores=NS)
    out_type = [jax.ShapeDtypeStruct((NC, N_PAD, D), jnp.float32)]
    if compute_cnt:
        out_type.append(jax.ShapeDtypeStruct((NC, N_PAD), jnp.float32))
    scratch = [
        pltpu.VMEM((G, WIN), jnp.int32),
        pltpu.VMEM((G, WIN), jnp.int32),
        pltpu.VMEM((G, WIN), jnp.int32),
        pltpu.VMEM((G, WIN), jnp.int32),
        pltpu.VMEM((WIN, D), jnp.float32),
        pltpu.VMEM((WIN, D), jnp.float32),
        pltpu.VMEM((WIN, D), jnp.float32),
        pltpu.VMEM((WIN, D), jnp.float32),
        pltpu.VMEM((WIN,), jnp.int32),
        pltpu.VMEM((WIN,), jnp.int32),
    ]
    if compute_cnt:
        scratch.append(pltpu.VMEM((WIN,), jnp.float32))
    scratch.append(pltpu.VMEM_SHARED((N_PAD, D), jnp.float32))
    if compute_cnt:
        scratch.append(pltpu.VMEM_SHARED((N_PAD,), jnp.float32))
    nsem = 10 + (4 if compute_cnt else 0)
    scratch += [pltpu.SemaphoreType.DMA] * nsem

    return pl.kernel(
        functools.partial(_sc_agg_body, compute_cnt),
        out_type=tuple(out_type),
        mesh=mesh,
        scratch_types=scratch,
        name=f"sage_sc_agg_cnt{int(compute_cnt)}",
    )


_SC_AGG_CNT = _make_sc_agg(True)
_SC_AGG = _make_sc_agg(False)


def _combine_body(relu, a, cc, xr, wn, ws, br, o):
    cnt = cc[0] + cc[1]
    r = 1.0 / jnp.maximum(cnt, 1.0)
    mean = (a[0] + a[1]) * r
    acc = jnp.dot(mean, wn[...], preferred_element_type=jnp.float32)
    acc = acc + jnp.dot(xr[...], ws[...], preferred_element_type=jnp.float32)
    acc = acc + br[...]
    if relu:
        acc = jnp.maximum(acc, 0.0)
    o[...] = acc


_R_BLK = 2000


def _combine(agg, cnt3, xr, wn, ws, br, relu):
    grid = (N_NODES // _R_BLK,)
    row_spec = pl.BlockSpec((_R_BLK, D), lambda i: (i, 0))
    a_spec = pl.BlockSpec((NC, _R_BLK, D), lambda i: (0, i, 0))
    c_spec = pl.BlockSpec((NC, _R_BLK, 1), lambda i: (0, i, 0))
    w_spec = pl.BlockSpec((D, D), lambda i: (0, 0))
    b_spec = pl.BlockSpec((1, D), lambda i: (0, 0))
    return pl.pallas_call(
        functools.partial(_combine_body, relu),
        grid=grid,
        in_specs=[a_spec, c_spec, row_spec, w_spec, w_spec, b_spec],
        out_specs=row_spec,
        out_shape=jax.ShapeDtypeStruct((N_NODES, D), jnp.float32),
        name=f"sage_combine_relu{int(relu)}",
    )(agg, cnt3, xr, wn, ws, br)


_PAD_N = E_PAD - N_EDGES
_PAD_SRC = jnp.asarray(np.arange(_PAD_N, dtype=np.int32) % N_NODES)
_PAD_DST = jnp.asarray(
    N_NODES + np.arange(_PAD_N, dtype=np.int32) % (N_PAD - N_NODES))


def kernel(x, edge_index, W1_self, W1_neigh, b1, W2_self, W2_neigh, b2):
    src = edge_index[0]
    dst = edge_index[1]
    src_p = jnp.concatenate([src, _PAD_SRC]).reshape(IDX_WINDOWS, WIN)
    dst_p = jnp.concatenate([dst, _PAD_DST]).reshape(IDX_WINDOWS, WIN)

    agg1, cnt = _SC_AGG_CNT(x, src_p, dst_p)
    cnt3 = cnt.reshape(NC, N_PAD, 1)
    b1r = b1.reshape(1, D)
    b2r = b2.reshape(1, D)

    h = _combine(agg1, cnt3, x, W1_neigh, W1_self, b1r, relu=True)
    (agg2,) = _SC_AGG(h, src_p, dst_p)
    out = _combine(agg2, cnt3, h, W2_neigh, W2_self, b2r, relu=False)
    return out


# speedup vs baseline: 14.6570x; 1.0103x over previous
import functools

import jax
import jax.numpy as jnp
import numpy as np
from jax import lax
from jax.experimental import pallas as pl
from jax.experimental.pallas import tpu as pltpu
from jax.experimental.pallas import tpu_sc as plsc

N_NODES = 10000
N_EDGES = 320000
D = 128

NC = 2
NS = 16
NW = NC * NS

WIN = 64
G = 8
GW = G * WIN
N_WINDOWS = 5120
WPW = N_WINDOWS // NW
NG = WPW // G
IDX_WINDOWS = N_WINDOWS + G
E_PAD = IDX_WINDOWS * WIN
N_PAD = 10240
RPT = N_PAD // NS
NBUF = 4
LAG = 2


def _sc_agg_body(compute_cnt, x_hbm, src_hbm, dst_hbm, *refs):
    if compute_cnt:
        (agg_out, cnt_out, sidx0, sidx1, didx0, didx1, d2d0, d2d1,
         rows0, rows1, rows2, rows3, pidx, pidx_lo, ones_v,
         agg_sh, cnt_sh,
         gsem0, gsem1, gsem2, gsem3, ssem0, ssem1, ssem2, ssem3,
         csem0, csem1, csem2, csem3, isem0, isem1) = refs
        csem = (csem0, csem1, csem2, csem3)
    else:
        (agg_out, sidx0, sidx1, didx0, didx1, d2d0, d2d1,
         rows0, rows1, rows2, rows3, pidx, pidx_lo,
         agg_sh,
         gsem0, gsem1, gsem2, gsem3, ssem0, ssem1, ssem2, ssem3,
         isem0, isem1) = refs

    sidx = (sidx0, sidx1)
    didx = (didx0, didx1)
    d2d = (d2d0, d2d1)
    rows = (rows0, rows1, rows2, rows3)
    gsem = (gsem0, gsem1, gsem2, gsem3)
    ssem = (ssem0, ssem1, ssem2, ssem3)
    isem = (isem0, isem1)

    c = lax.axis_index("c")
    s = lax.axis_index("s")
    wid = s * NC + c
    ebase = wid * WPW * WIN

    z16 = jnp.zeros((16,), jnp.float32)
    iota16 = lax.iota(jnp.int32, 16)
    for j in range(WIN // 16):
        pidx[pl.ds(j * 16, 16)] = iota16 + (
            N_NODES + (s * WIN + j * 16) % (N_PAD - N_NODES - 16))
        pidx_lo[pl.ds(j * 16, 16)] = iota16 + s * WIN + j * 16

    def _zrow(i, carry):
        for j in range(D // 16):
            rows0[i, pl.ds(j * 16, 16)] = z16
        return carry

    lax.fori_loop(0, WIN, _zrow, 0)

    if compute_cnt:
        one16 = jnp.ones((16,), jnp.float32)
        for j in range(WIN // 16):
            ones_v[pl.ds(j * 16, 16)] = one16

    base = s * RPT
    for t in range(RPT // WIN):
        pltpu.sync_copy(rows0, agg_sh.at[pl.ds(base + t * WIN, WIN)])
    if compute_cnt:
        for t in range(RPT // D):
            pltpu.sync_copy(rows0.at[0], cnt_sh.at[pl.ds(base + t * D, D)])
    plsc.subcore_barrier()

    def wait_gather(b):
        pltpu.make_async_copy(x_hbm.at[pidx_lo], rows[b], gsem[b]).wait()

    def wait_scatter(b):
        pltpu.make_async_copy(rows[b], agg_sh.at[pidx], ssem[b]).wait()
        if compute_cnt:
            pltpu.make_async_copy(ones_v, cnt_sh.at[pidx], csem[b]).wait()

    def wait_idx(gb, eoff):
        pltpu.make_async_copy(src_hbm.at[pl.ds(eoff, GW)], sidx[gb],
                              isem[gb]).wait()
        pltpu.make_async_copy(dst_hbm.at[pl.ds(eoff, GW)], didx[gb],
                              isem[gb]).wait()

    def fill_d2d(gb):
        for k in range(G):
            for j in range(WIN // 16):
                d2d[gb][k, pl.ds(j * 16, 16)] = (
                    didx[gb][pl.ds(k * WIN + j * 16, 16)])

    pltpu.async_copy(src_hbm.at[pl.ds(ebase, GW)], sidx0, isem0)
    pltpu.async_copy(dst_hbm.at[pl.ds(ebase, GW)], didx0, isem0)
    for j in range(WIN // 16):
        d2d1[G - 2, pl.ds(j * 16, 16)] = pidx[pl.ds(j * 16, 16)]
        d2d1[G - 1, pl.ds(j * 16, 16)] = pidx[pl.ds(j * 16, 16)]
    pltpu.async_copy(x_hbm.at[pidx_lo], rows2, gsem2)
    pltpu.async_copy(x_hbm.at[pidx_lo], rows3, gsem3)
    for b in range(LAG):
        pltpu.async_copy(rows[b], agg_sh.at[pidx], ssem[b], add=True)
        if compute_cnt:
            pltpu.async_copy(ones_v, cnt_sh.at[pidx], csem[b], add=True)

    def pair_body(gg, carry):
        for gpar in range(2):
            gb = gpar
            g = 2 * gg + gpar
            geoff = ebase + g * GW
            wait_idx(gb, geoff)
            fill_d2d(gb)
            for k in range(G):
                b = k % NBUF
                bv = (k + LAG) % NBUF
                wait_scatter(b)
                pltpu.async_copy(
                    x_hbm.at[sidx[gb].at[pl.ds(k * WIN, WIN)]],
                    rows[b], gsem[b])
                wait_gather(bv)
                if k < LAG:
                    dv = d2d[gb ^ 1].at[G - LAG + k]
                else:
                    dv = d2d[gb].at[k - LAG]
                pltpu.async_copy(rows[bv], agg_sh.at[dv], ssem[bv], add=True)
                if compute_cnt:
                    pltpu.async_copy(ones_v, cnt_sh.at[dv], csem[bv],
                                     add=True)
                if k == 3:
                    nxt = ebase + (g + 1) * GW
                    pltpu.async_copy(src_hbm.at[pl.ds(nxt, GW)], sidx[gb ^ 1],
                                     isem[gb ^ 1])
                    pltpu.async_copy(dst_hbm.at[pl.ds(nxt, GW)], didx[gb ^ 1],
                                     isem[gb ^ 1])
        return carry

    lax.fori_loop(0, NG // 2, pair_body, 0)

    for e in range(LAG):
        kv = G - LAG + e
        bv = (WPW - LAG + e) % NBUF
        wait_gather(bv)
        pltpu.async_copy(rows[bv], agg_sh.at[d2d1.at[kv]], ssem[bv],
                         add=True)
        if compute_cnt:
            pltpu.async_copy(ones_v, cnt_sh.at[d2d1.at[kv]], csem[bv],
                             add=True)
    for b in range(NBUF):
        wait_scatter(b)
    wait_idx(0, ebase + NG * GW)

    plsc.subcore_barrier()

    for t in range(RPT // 128):
        pltpu.sync_copy(agg_sh.at[pl.ds(base + t * 128, 128)],
                        agg_out.at[c, pl.ds(base + t * 128, 128)])
    if compute_cnt:
        pltpu.sync_copy(cnt_sh.at[pl.ds(base, RPT)],
                        cnt_out.at[c, pl.ds(base, RPT)])


def _make_sc_agg(compute_cnt):
    mesh = plsc.VectorSubcoreMesh(core_axis_name="c", subcore_axis_name="s",
                                  num_cores=NC, num_subcores=NS)
    out_type = [jax.ShapeDtypeStruct((NC, N_PAD, D), jnp.float32)]
    if compute_cnt:
        out_type.append(jax.ShapeDtypeStruct((NC, N_PAD), jnp.float32))
    scratch = [
        pltpu.VMEM((GW,), jnp.int32),
        pltpu.VMEM((GW,), jnp.int32),
        pltpu.VMEM((GW,), jnp.int32),
        pltpu.VMEM((GW,), jnp.int32),
        pltpu.VMEM((G, WIN), jnp.int32),
        pltpu.VMEM((G, WIN), jnp.int32),
        pltpu.VMEM((WIN, D), jnp.float32),
        pltpu.VMEM((WIN, D), jnp.float32),
        pltpu.VMEM((WIN, D), jnp.float32),
        pltpu.VMEM((WIN, D), jnp.float32),
        pltpu.VMEM((WIN,), jnp.int32),
        pltpu.VMEM((WIN,), jnp.int32),
    ]
    if compute_cnt:
        scratch.append(pltpu.VMEM((WIN,), jnp.float32))
    scratch.append(pltpu.VMEM_SHARED((N_PAD, D), jnp.float32))
    if compute_cnt:
        scratch.append(pltpu.VMEM_SHARED((N_PAD,), jnp.float32))
    nsem = 10 + (4 if compute_cnt else 0)
    scratch += [pltpu.SemaphoreType.DMA] * nsem

    return pl.kernel(
        functools.partial(_sc_agg_body, compute_cnt),
        out_type=tuple(out_type),
        mesh=mesh,
        scratch_types=scratch,
        name=f"sage_sc_agg_cnt{int(compute_cnt)}",
    )


_SC_AGG_CNT = _make_sc_agg(True)
_SC_AGG = _make_sc_agg(False)


def _combine_body(relu, a, cc, xr, wn, ws, br, o):
    cnt = cc[0] + cc[1]
    r = 1.0 / jnp.maximum(cnt, 1.0)
    mean = (a[0] + a[1]) * r
    acc = jnp.dot(mean, wn[...], preferred_element_type=jnp.float32)
    acc = acc + jnp.dot(xr[...], ws[...], preferred_element_type=jnp.float32)
    acc = acc + br[...]
    if relu:
        acc = jnp.maximum(acc, 0.0)
    o[...] = acc


_R_BLK = 2000


def _combine(agg, cnt3, xr, wn, ws, br, relu):
    grid = (N_NODES // _R_BLK,)
    row_spec = pl.BlockSpec((_R_BLK, D), lambda i: (i, 0))
    a_spec = pl.BlockSpec((NC, _R_BLK, D), lambda i: (0, i, 0))
    c_spec = pl.BlockSpec((NC, _R_BLK, 1), lambda i: (0, i, 0))
    w_spec = pl.BlockSpec((D, D), lambda i: (0, 0))
    b_spec = pl.BlockSpec((1, D), lambda i: (0, 0))
    return pl.pallas_call(
        functools.partial(_combine_body, relu),
        grid=grid,
        in_specs=[a_spec, c_spec, row_spec, w_spec, w_spec, b_spec],
        out_specs=row_spec,
        out_shape=jax.ShapeDtypeStruct((N_NODES, D), jnp.float32),
        name=f"sage_combine_relu{int(relu)}",
    )(agg, cnt3, xr, wn, ws, br)


_PAD_N = E_PAD - N_EDGES
_PAD_SRC = jnp.asarray(np.arange(_PAD_N, dtype=np.int32) % N_NODES)
_PAD_DST = jnp.asarray(
    N_NODES + np.arange(_PAD_N, dtype=np.int32) % (N_PAD - N_NODES))


def kernel(x, edge_index, W1_self, W1_neigh, b1, W2_self, W2_neigh, b2):
    src = edge_index[0]
    dst = edge_index[1]
    src_p = jnp.concatenate([src, _PAD_SRC])
    dst_p = jnp.concatenate([dst, _PAD_DST])

    agg1, cnt = _SC_AGG_CNT(x, src_p, dst_p)
    cnt3 = cnt.reshape(NC, N_PAD, 1)
    b1r = b1.reshape(1, D)
    b2r = b2.reshape(1, D)

    h = _combine(agg1, cnt3, x, W1_neigh, W1_self, b1r, relu=True)
    (agg2,) = _SC_AGG(h, src_p, dst_p)
    out = _combine(agg2, cnt3, h, W2_neigh, W2_self, b2r, relu=False)
    return out


# speedup vs baseline: 15.1918x; 1.0365x over previous
import functools

import jax
import jax.numpy as jnp
import numpy as np
from jax import lax
from jax.experimental import pallas as pl
from jax.experimental.pallas import tpu as pltpu
from jax.experimental.pallas import tpu_sc as plsc

N_NODES = 10000
N_EDGES = 320000
D = 128

NC = 2
NS = 16
NW = NC * NS

WIN = 64
G = 8
GW = G * WIN
N_WINDOWS = 5120
WPW = N_WINDOWS // NW
NG = WPW // G
IDX_WINDOWS = N_WINDOWS + G
E_PAD = IDX_WINDOWS * WIN
N_PAD = 10240
RPT = N_PAD // NS
NBUF = 4
LAG = 2


def _sc_agg_body(compute_cnt, x_hbm, src_hbm, dst_hbm, *refs):
    if compute_cnt:
        (agg_out, cnt_out, sidx0, sidx1, didx0, didx1, d2d0, d2d1,
         rows0, rows1, rows2, rows3, pidx, pidx_lo, ones_v,
         agg_sh, cnt_sh,
         gsem0, gsem1, gsem2, gsem3, ssem0, ssem1, ssem2, ssem3,
         csem0, csem1, csem2, csem3, isem0, isem1) = refs
        csem = (csem0, csem1, csem2, csem3)
    else:
        (agg_out, sidx0, sidx1, didx0, didx1, d2d0, d2d1,
         rows0, rows1, rows2, rows3, pidx, pidx_lo,
         agg_sh,
         gsem0, gsem1, gsem2, gsem3, ssem0, ssem1, ssem2, ssem3,
         isem0, isem1) = refs

    sidx = (sidx0, sidx1)
    didx = (didx0, didx1)
    d2d = (d2d0, d2d1)
    rows = (rows0, rows1, rows2, rows3)
    gsem = (gsem0, gsem1, gsem2, gsem3)
    ssem = (ssem0, ssem1, ssem2, ssem3)
    isem = (isem0, isem1)

    c = lax.axis_index("c")
    s = lax.axis_index("s")
    wid = s * NC + c
    ebase = wid * WPW * WIN

    z16 = jnp.zeros((16,), jnp.float32)
    iota16 = lax.iota(jnp.int32, 16)
    for j in range(WIN // 16):
        pidx[pl.ds(j * 16, 16)] = iota16 + (
            N_NODES + (s * WIN + j * 16) % (N_PAD - N_NODES - 16))
        pidx_lo[pl.ds(j * 16, 16)] = iota16 + s * WIN + j * 16

    def _zrow(i, carry):
        for j in range(D // 16):
            rows0[i, pl.ds(j * 16, 16)] = z16
        return carry

    lax.fori_loop(0, WIN, _zrow, 0)

    if compute_cnt:
        one16 = jnp.ones((16,), jnp.float32)
        for j in range(WIN // 16):
            ones_v[pl.ds(j * 16, 16)] = one16

    base = s * RPT
    for t in range(RPT // WIN):
        pltpu.sync_copy(rows0, agg_sh.at[pl.ds(base + t * WIN, WIN)])
    if compute_cnt:
        for t in range(RPT // D):
            pltpu.sync_copy(rows0.at[0], cnt_sh.at[pl.ds(base + t * D, D)])
    plsc.subcore_barrier()

    def wait_gather(b):
        pltpu.make_async_copy(x_hbm.at[pidx_lo], rows[b], gsem[b]).wait()

    def wait_scatter(b):
        pltpu.make_async_copy(rows[b], agg_sh.at[pidx], ssem[b]).wait()
        if compute_cnt:
            pltpu.make_async_copy(ones_v, cnt_sh.at[pidx], csem[b]).wait()

    def wait_idx(gb, eoff):
        pltpu.make_async_copy(src_hbm.at[pl.ds(eoff, GW)], sidx[gb],
                              isem[gb]).wait()
        pltpu.make_async_copy(dst_hbm.at[pl.ds(eoff, GW)], didx[gb],
                              isem[gb]).wait()

    def fill_d2d(gb):
        for k in range(G):
            for j in range(WIN // 16):
                d2d[gb][k, pl.ds(j * 16, 16)] = (
                    didx[gb][pl.ds(k * WIN + j * 16, 16)])

    pltpu.async_copy(src_hbm.at[pl.ds(ebase, GW)], sidx0, isem0)
    pltpu.async_copy(dst_hbm.at[pl.ds(ebase, GW)], didx0, isem0)
    for j in range(WIN // 16):
        d2d1[G - 2, pl.ds(j * 16, 16)] = pidx[pl.ds(j * 16, 16)]
        d2d1[G - 1, pl.ds(j * 16, 16)] = pidx[pl.ds(j * 16, 16)]
    pltpu.async_copy(x_hbm.at[pidx_lo], rows2, gsem2)
    pltpu.async_copy(x_hbm.at[pidx_lo], rows3, gsem3)
    for b in range(LAG):
        pltpu.async_copy(rows[b], agg_sh.at[pidx], ssem[b], add=True)
        if compute_cnt:
            pltpu.async_copy(ones_v, cnt_sh.at[pidx], csem[b], add=True)

    def pair_body(gg, carry):
        for gpar in range(2):
            gb = gpar
            g = 2 * gg + gpar
            geoff = ebase + g * GW
            wait_idx(gb, geoff)
            fill_d2d(gb)
            for k in range(G):
                b = k % NBUF
                bv = (k + LAG) % NBUF
                wait_scatter(b)
                pltpu.async_copy(
                    x_hbm.at[sidx[gb].at[pl.ds(k * WIN, WIN)]],
                    rows[b], gsem[b])
                wait_gather(bv)
                if k < LAG:
                    dv = d2d[gb ^ 1].at[G - LAG + k]
                else:
                    dv = d2d[gb].at[k - LAG]
                pltpu.async_copy(rows[bv], agg_sh.at[dv], ssem[bv], add=True)
                if compute_cnt:
                    pltpu.async_copy(ones_v, cnt_sh.at[dv], csem[bv],
                                     add=True)
                if k == 3:
                    nxt = ebase + (g + 1) * GW
                    pltpu.async_copy(src_hbm.at[pl.ds(nxt, GW)], sidx[gb ^ 1],
                                     isem[gb ^ 1])
                    pltpu.async_copy(dst_hbm.at[pl.ds(nxt, GW)], didx[gb ^ 1],
                                     isem[gb ^ 1])
        return carry

    lax.fori_loop(0, NG // 2, pair_body, 0)

    for e in range(LAG):
        kv = G - LAG + e
        bv = (WPW - LAG + e) % NBUF
        wait_gather(bv)
        pltpu.async_copy(rows[bv], agg_sh.at[d2d1.at[kv]], ssem[bv],
                         add=True)
        if compute_cnt:
            pltpu.async_copy(ones_v, cnt_sh.at[d2d1.at[kv]], csem[bv],
                             add=True)
    for b in range(NBUF):
        wait_scatter(b)
    wait_idx(0, ebase + NG * GW)

    plsc.subcore_barrier()

    for t in range(RPT // 128):
        pltpu.sync_copy(agg_sh.at[pl.ds(base + t * 128, 128)],
                        agg_out.at[c, pl.ds(base + t * 128, 128)])
    if compute_cnt:
        pltpu.sync_copy(cnt_sh.at[pl.ds(base, RPT)],
                        cnt_out.at[c, pl.ds(base, RPT)])


def _make_sc_agg(compute_cnt):
    mesh = plsc.VectorSubcoreMesh(core_axis_name="c", subcore_axis_name="s",
                                  num_cores=NC, num_subcores=NS)
    out_type = [jax.ShapeDtypeStruct((NC, N_PAD, D), jnp.float32)]
    if compute_cnt:
        out_type.append(jax.ShapeDtypeStruct((NC, N_PAD), jnp.float32))
    scratch = [
        pltpu.VMEM((GW,), jnp.int32),
        pltpu.VMEM((GW,), jnp.int32),
        pltpu.VMEM((GW,), jnp.int32),
        pltpu.VMEM((GW,), jnp.int32),
        pltpu.VMEM((G, WIN), jnp.int32),
        pltpu.VMEM((G, WIN), jnp.int32),
        pltpu.VMEM((WIN, D), jnp.float32),
        pltpu.VMEM((WIN, D), jnp.float32),
        pltpu.VMEM((WIN, D), jnp.float32),
        pltpu.VMEM((WIN, D), jnp.float32),
        pltpu.VMEM((WIN,), jnp.int32),
        pltpu.VMEM((WIN,), jnp.int32),
    ]
    if compute_cnt:
        scratch.append(pltpu.VMEM((WIN,), jnp.float32))
    scratch.append(pltpu.VMEM_SHARED((N_PAD, D), jnp.float32))
    if compute_cnt:
        scratch.append(pltpu.VMEM_SHARED((N_PAD,), jnp.float32))
    nsem = 10 + (4 if compute_cnt else 0)
    scratch += [pltpu.SemaphoreType.DMA] * nsem

    return pl.kernel(
        functools.partial(_sc_agg_body, compute_cnt),
        out_type=tuple(out_type),
        mesh=mesh,
        scratch_types=scratch,
        name=f"sage_sc_agg_cnt{int(compute_cnt)}",
    )


_SC_AGG_CNT = _make_sc_agg(True)
_SC_AGG = _make_sc_agg(False)


def _combine_body(relu, a, cc, xr, wn, ws, br, o):
    i = pl.program_id(0)
    cnt = (cc[0:1, pl.ds(i * _R_BLK, _R_BLK)]
           + cc[1:2, pl.ds(i * _R_BLK, _R_BLK)])
    r = 1.0 / jnp.maximum(cnt, 1.0)
    r_col = jnp.transpose(r, (1, 0))
    mean = (a[0] + a[1]) * r_col
    acc = jnp.dot(mean, wn[...], preferred_element_type=jnp.float32)
    acc = acc + jnp.dot(xr[...], ws[...], preferred_element_type=jnp.float32)
    acc = acc + br[...]
    if relu:
        acc = jnp.maximum(acc, 0.0)
    o[...] = acc


_R_BLK = 2048


def _combine(agg, cnt2, xr, wn, ws, br, relu):
    grid = (N_PAD // _R_BLK,)
    row_spec = pl.BlockSpec((_R_BLK, D), lambda i: (i, 0))
    a_spec = pl.BlockSpec((NC, _R_BLK, D), lambda i: (0, i, 0))
    c_spec = pl.BlockSpec((NC, N_PAD), lambda i: (0, 0))
    w_spec = pl.BlockSpec((D, D), lambda i: (0, 0))
    b_spec = pl.BlockSpec((1, D), lambda i: (0, 0))
    return pl.pallas_call(
        functools.partial(_combine_body, relu),
        grid=grid,
        in_specs=[a_spec, c_spec, row_spec, w_spec, w_spec, b_spec],
        out_specs=row_spec,
        out_shape=jax.ShapeDtypeStruct((N_NODES, D), jnp.float32),
        name=f"sage_combine_relu{int(relu)}",
    )(agg, cnt2, xr, wn, ws, br)


_PAD_N = E_PAD - N_EDGES
_PAD_SRC = jnp.asarray(np.arange(_PAD_N, dtype=np.int32) % N_NODES)
_PAD_DST = jnp.asarray(
    N_NODES + np.arange(_PAD_N, dtype=np.int32) % (N_PAD - N_NODES))


def kernel(x, edge_index, W1_self, W1_neigh, b1, W2_self, W2_neigh, b2):
    src = edge_index[0]
    dst = edge_index[1]
    src_p = jnp.concatenate([src, _PAD_SRC])
    dst_p = jnp.concatenate([dst, _PAD_DST])

    agg1, cnt = _SC_AGG_CNT(x, src_p, dst_p)
    b1r = b1.reshape(1, D)
    b2r = b2.reshape(1, D)

    h = _combine(agg1, cnt, x, W1_neigh, W1_self, b1r, relu=True)
    (agg2,) = _SC_AGG(h, src_p, dst_p)
    out = _combine(agg2, cnt, h, W2_neigh, W2_self, b2r, relu=False)
    return out


# speedup vs baseline: 15.8025x; 1.0402x over previous
import functools

import jax
import jax.numpy as jnp
import numpy as np
from jax import lax
from jax.experimental import pallas as pl
from jax.experimental.pallas import tpu as pltpu
from jax.experimental.pallas import tpu_sc as plsc

N_NODES = 10000
N_EDGES = 320000
D = 128

NC = 2
NS = 16
NW = NC * NS

WIN = 64
G = 8
GW = G * WIN
N_WINDOWS = 5120
WPW = N_WINDOWS // NW
NG = WPW // G
IDX_WINDOWS = N_WINDOWS + G
E_PAD = IDX_WINDOWS * WIN
N_PAD = 10240
RPT = N_PAD // NS
NBUF = 4
LAG = 2


def _sc_agg_body(compute_cnt, x_hbm, src_hbm, dst_hbm, *refs):
    if compute_cnt:
        (agg_out, cnt_out, sidx0, sidx1, didx0, didx1, d2d0, d2d1,
         rows0, rows1, rows2, rows3, pidx, pidx_lo, ones_v,
         agg_sh, cnt_sh,
         gsem0, gsem1, gsem2, gsem3, ssem0, ssem1, ssem2, ssem3,
         csem0, csem1, csem2, csem3, isem0, isem1) = refs
        csem = (csem0, csem1, csem2, csem3)
    else:
        (agg_out, sidx0, sidx1, didx0, didx1, d2d0, d2d1,
         rows0, rows1, rows2, rows3, pidx, pidx_lo,
         agg_sh,
         gsem0, gsem1, gsem2, gsem3, ssem0, ssem1, ssem2, ssem3,
         isem0, isem1) = refs

    sidx = (sidx0, sidx1)
    didx = (didx0, didx1)
    d2d = (d2d0, d2d1)
    rows = (rows0, rows1, rows2, rows3)
    gsem = (gsem0, gsem1, gsem2, gsem3)
    ssem = (ssem0, ssem1, ssem2, ssem3)
    isem = (isem0, isem1)

    c = lax.axis_index("c")
    s = lax.axis_index("s")
    wid = s * NC + c
    ebase = wid * WPW * WIN

    z16 = jnp.zeros((16,), jnp.float32)
    iota16 = lax.iota(jnp.int32, 16)
    for j in range(WIN // 16):
        pidx[pl.ds(j * 16, 16)] = iota16 + (
            N_NODES + (s * WIN + j * 16) % (N_PAD - N_NODES - 16))
        pidx_lo[pl.ds(j * 16, 16)] = iota16 + s * WIN + j * 16

    def _zrow(i, carry):
        for j in range(D // 16):
            rows0[i, pl.ds(j * 16, 16)] = z16
        return carry

    lax.fori_loop(0, WIN, _zrow, 0)

    if compute_cnt:
        one16 = jnp.ones((16,), jnp.float32)
        for j in range(WIN // 16):
            ones_v[pl.ds(j * 16, 16)] = one16

    base = s * RPT
    for t in range(RPT // WIN):
        pltpu.sync_copy(rows0, agg_sh.at[pl.ds(base + t * WIN, WIN)])
    if compute_cnt:
        for t in range(RPT // D):
            pltpu.sync_copy(rows0.at[0], cnt_sh.at[pl.ds(base + t * D, D)])
    plsc.subcore_barrier()

    def wait_gather(b):
        pltpu.make_async_copy(x_hbm.at[pidx_lo], rows[b], gsem[b]).wait()

    def wait_scatter(b):
        pltpu.make_async_copy(rows[b], agg_sh.at[pidx], ssem[b]).wait()
        if compute_cnt:
            pltpu.make_async_copy(ones_v, cnt_sh.at[pidx], csem[b]).wait()

    def wait_idx(gb, eoff):
        pltpu.make_async_copy(src_hbm.at[pl.ds(eoff, GW)], sidx[gb],
                              isem[gb]).wait()
        pltpu.make_async_copy(dst_hbm.at[pl.ds(eoff, GW)], didx[gb],
                              isem[gb]).wait()

    def fill_d2d(gb):
        for k in range(G):
            for j in range(WIN // 16):
                d2d[gb][k, pl.ds(j * 16, 16)] = (
                    didx[gb][pl.ds(k * WIN + j * 16, 16)])

    pltpu.async_copy(src_hbm.at[pl.ds(ebase, GW)], sidx0, isem0)
    pltpu.async_copy(dst_hbm.at[pl.ds(ebase, GW)], didx0, isem0)
    for j in range(WIN // 16):
        d2d1[G - 2, pl.ds(j * 16, 16)] = pidx[pl.ds(j * 16, 16)]
        d2d1[G - 1, pl.ds(j * 16, 16)] = pidx[pl.ds(j * 16, 16)]
    pltpu.async_copy(x_hbm.at[pidx_lo], rows2, gsem2)
    pltpu.async_copy(x_hbm.at[pidx_lo], rows3, gsem3)
    for b in range(LAG):
        pltpu.async_copy(rows[b], agg_sh.at[pidx], ssem[b], add=True)
        if compute_cnt:
            pltpu.async_copy(ones_v, cnt_sh.at[pidx], csem[b], add=True)

    def pair_body(gg, carry):
        for gpar in range(2):
            gb = gpar
            g = 2 * gg + gpar
            geoff = ebase + g * GW
            wait_idx(gb, geoff)
            fill_d2d(gb)
            for k in range(G):
                b = k % NBUF
                bv = (k + LAG) % NBUF
                wait_scatter(b)
                pltpu.async_copy(
                    x_hbm.at[sidx[gb].at[pl.ds(k * WIN, WIN)]],
                    rows[b], gsem[b])
                wait_gather(bv)
                if k < LAG:
                    dv = d2d[gb ^ 1].at[G - LAG + k]
                else:
                    dv = d2d[gb].at[k - LAG]
                pltpu.async_copy(rows[bv], agg_sh.at[dv], ssem[bv], add=True)
                if compute_cnt:
                    pltpu.async_copy(ones_v, cnt_sh.at[dv], csem[bv],
                                     add=True)
                if k == 3:
                    nxt = ebase + (g + 1) * GW
                    pltpu.async_copy(src_hbm.at[pl.ds(nxt, GW)], sidx[gb ^ 1],
                                     isem[gb ^ 1])
                    pltpu.async_copy(dst_hbm.at[pl.ds(nxt, GW)], didx[gb ^ 1],
                                     isem[gb ^ 1])
        return carry

    lax.fori_loop(0, NG // 2, pair_body, 0)

    for e in range(LAG):
        kv = G - LAG + e
        bv = (WPW - LAG + e) % NBUF
        wait_gather(bv)
        pltpu.async_copy(rows[bv], agg_sh.at[d2d1.at[kv]], ssem[bv],
                         add=True)
        if compute_cnt:
            pltpu.async_copy(ones_v, cnt_sh.at[d2d1.at[kv]], csem[bv],
                             add=True)
    for b in range(NBUF):
        wait_scatter(b)
    wait_idx(0, ebase + NG * GW)

    plsc.subcore_barrier()

    for t in range(RPT // 128):
        pltpu.sync_copy(agg_sh.at[pl.ds(base + t * 128, 128)],
                        agg_out.at[c, pl.ds(base + t * 128, 128)])
    if compute_cnt:
        pltpu.sync_copy(cnt_sh.at[pl.ds(base, RPT)],
                        cnt_out.at[c, pl.ds(base, RPT)])


def _make_sc_agg(compute_cnt):
    mesh = plsc.VectorSubcoreMesh(core_axis_name="c", subcore_axis_name="s",
                                  num_cores=NC, num_subcores=NS)
    out_type = [jax.ShapeDtypeStruct((NC, N_PAD, D), jnp.float32)]
    if compute_cnt:
        out_type.append(jax.ShapeDtypeStruct((NC, N_PAD), jnp.float32))
    scratch = [
        pltpu.VMEM((GW,), jnp.int32),
        pltpu.VMEM((GW,), jnp.int32),
        pltpu.VMEM((GW,), jnp.int32),
        pltpu.VMEM((GW,), jnp.int32),
        pltpu.VMEM((G, WIN), jnp.int32),
        pltpu.VMEM((G, WIN), jnp.int32),
        pltpu.VMEM((WIN, D), jnp.float32),
        pltpu.VMEM((WIN, D), jnp.float32),
        pltpu.VMEM((WIN, D), jnp.float32),
        pltpu.VMEM((WIN, D), jnp.float32),
        pltpu.VMEM((WIN,), jnp.int32),
        pltpu.VMEM((WIN,), jnp.int32),
    ]
    if compute_cnt:
        scratch.append(pltpu.VMEM((WIN,), jnp.float32))
    scratch.append(pltpu.VMEM_SHARED((N_PAD, D), jnp.float32))
    if compute_cnt:
        scratch.append(pltpu.VMEM_SHARED((N_PAD,), jnp.float32))
    nsem = 10 + (4 if compute_cnt else 0)
    scratch += [pltpu.SemaphoreType.DMA] * nsem

    return pl.kernel(
        functools.partial(_sc_agg_body, compute_cnt),
        out_type=tuple(out_type),
        mesh=mesh,
        scratch_types=scratch,
        name=f"sage_sc_agg_cnt{int(compute_cnt)}",
    )


_SC_AGG_CNT = _make_sc_agg(True)
_SC_AGG = _make_sc_agg(False)


def _combine_body(relu, a, cc, xr, wn, ws, br, o):
    i = pl.program_id(0)
    cnt = (cc[0:1, pl.ds(i * _R_BLK, _R_BLK)]
           + cc[1:2, pl.ds(i * _R_BLK, _R_BLK)])
    r = 1.0 / jnp.maximum(cnt, 1.0)
    r_col = jnp.transpose(r, (1, 0))
    mean = (a[0] + a[1]) * r_col
    acc = jnp.dot(mean, wn[...], preferred_element_type=jnp.float32)
    acc = acc + jnp.dot(xr[...], ws[...], preferred_element_type=jnp.float32)
    acc = acc + br[...]
    if relu:
        acc = jnp.maximum(acc, 0.0)
    o[...] = acc


_R_BLK = 2048


def _combine(agg, cnt2, xr, wn, ws, br, relu):
    grid = (N_PAD // _R_BLK,)
    row_spec = pl.BlockSpec((_R_BLK, D), lambda i: (i, 0))
    a_spec = pl.BlockSpec((NC, _R_BLK, D), lambda i: (0, i, 0))
    c_spec = pl.BlockSpec((NC, N_PAD), lambda i: (0, 0))
    w_spec = pl.BlockSpec((D, D), lambda i: (0, 0))
    b_spec = pl.BlockSpec((1, D), lambda i: (0, 0))
    return pl.pallas_call(
        functools.partial(_combine_body, relu),
        grid=grid,
        in_specs=[a_spec, c_spec, row_spec, w_spec, w_spec, b_spec],
        out_specs=row_spec,
        out_shape=jax.ShapeDtypeStruct((N_NODES, D), jnp.float32),
        name=f"sage_combine_relu{int(relu)}",
    )(agg, cnt2, xr, wn, ws, br)


_PAD_N = E_PAD - N_EDGES
_PAD_SRC = jnp.asarray(np.arange(_PAD_N, dtype=np.int32) % N_NODES)
_PAD_DST = jnp.asarray(
    N_NODES + np.arange(_PAD_N, dtype=np.int32) % (N_PAD - N_NODES))


def kernel(x, edge_index, W1_self, W1_neigh, b1, W2_self, W2_neigh, b2):
    flat = lax.optimization_barrier(jnp.reshape(edge_index, (2 * N_EDGES,)))
    src_p = jnp.concatenate([flat[:N_EDGES], _PAD_SRC])
    dst_p = jnp.concatenate([flat[N_EDGES:], _PAD_DST])

    agg1, cnt = _SC_AGG_CNT(x, src_p, dst_p)
    b1r = b1.reshape(1, D)
    b2r = b2.reshape(1, D)

    h = _combine(agg1, cnt, x, W1_neigh, W1_self, b1r, relu=True)
    (agg2,) = _SC_AGG(h, src_p, dst_p)
    out = _combine(agg2, cnt, h, W2_neigh, W2_self, b2r, relu=False)
    return out


# speedup vs baseline: 15.9440x; 1.0090x over previous
import functools

import jax
import jax.numpy as jnp
import numpy as np
from jax import lax
from jax.experimental import pallas as pl
from jax.experimental.pallas import tpu as pltpu
from jax.experimental.pallas import tpu_sc as plsc

N_NODES = 10000
N_EDGES = 320000
D = 128

NC = 2
NS = 16
NW = NC * NS

WIN = 64
G = 8
GW = G * WIN
N_WINDOWS = 5120
WPW = N_WINDOWS // NW
NG = WPW // G
IDX_WINDOWS = N_WINDOWS + G
E_PAD = IDX_WINDOWS * WIN
N_PAD = 10240
RPT = N_PAD // NS
NBUF = 4
LAG = 2


def _sc_agg_body(compute_cnt, x_hbm, src_hbm, dst_hbm, *refs):
    if compute_cnt:
        (agg_out, cnt_out, sidx0, sidx1, didx0, didx1, d2d0, d2d1,
         rows0, rows1, rows2, rows3, pidx, pidx_lo, ones_v,
         agg_sh, cnt_sh,
         gsem0, gsem1, gsem2, gsem3, ssem0, ssem1, ssem2, ssem3,
         csem0, csem1, csem2, csem3, isem0, isem1) = refs
        csem = (csem0, csem1, csem2, csem3)
    else:
        (agg_out, sidx0, sidx1, didx0, didx1, d2d0, d2d1,
         rows0, rows1, rows2, rows3, pidx, pidx_lo,
         agg_sh,
         gsem0, gsem1, gsem2, gsem3, ssem0, ssem1, ssem2, ssem3,
         isem0, isem1) = refs

    sidx = (sidx0, sidx1)
    didx = (didx0, didx1)
    d2d = (d2d0, d2d1)
    rows = (rows0, rows1, rows2, rows3)
    gsem = (gsem0, gsem1, gsem2, gsem3)
    ssem = (ssem0, ssem1, ssem2, ssem3)
    isem = (isem0, isem1)

    c = lax.axis_index("c")
    s = lax.axis_index("s")
    wid = s * NC + c
    ebase = wid * WPW * WIN

    z16 = jnp.zeros((16,), jnp.float32)
    iota16 = lax.iota(jnp.int32, 16)
    for j in range(WIN // 16):
        pidx[pl.ds(j * 16, 16)] = iota16 + (
            N_NODES + (s * WIN + j * 16) % (N_PAD - N_NODES - 16))
        pidx_lo[pl.ds(j * 16, 16)] = iota16 + s * WIN + j * 16

    def _zrow(i, carry):
        for j in range(D // 16):
            rows0[i, pl.ds(j * 16, 16)] = z16
        return carry

    lax.fori_loop(0, WIN, _zrow, 0)

    if compute_cnt:
        one16 = jnp.ones((16,), jnp.float32)
        for j in range(WIN // 16):
            ones_v[pl.ds(j * 16, 16)] = one16

    pltpu.async_copy(src_hbm.at[pl.ds(ebase, GW)], sidx0, isem0)
    pltpu.async_copy(dst_hbm.at[pl.ds(ebase, GW)], didx0, isem0)
    pltpu.async_copy(x_hbm.at[pidx_lo], rows2, gsem2)
    pltpu.async_copy(x_hbm.at[pidx_lo], rows3, gsem3)

    base = s * RPT
    for t in range(RPT // WIN):
        pltpu.sync_copy(rows0, agg_sh.at[pl.ds(base + t * WIN, WIN)])
    if compute_cnt:
        for t in range(RPT // D):
            pltpu.sync_copy(rows0.at[0], cnt_sh.at[pl.ds(base + t * D, D)])
    plsc.subcore_barrier()

    def wait_gather(b):
        pltpu.make_async_copy(x_hbm.at[pidx_lo], rows[b], gsem[b]).wait()

    def wait_scatter(b):
        pltpu.make_async_copy(rows[b], agg_sh.at[pidx], ssem[b]).wait()
        if compute_cnt:
            pltpu.make_async_copy(ones_v, cnt_sh.at[pidx], csem[b]).wait()

    def wait_idx(gb, eoff):
        pltpu.make_async_copy(src_hbm.at[pl.ds(eoff, GW)], sidx[gb],
                              isem[gb]).wait()
        pltpu.make_async_copy(dst_hbm.at[pl.ds(eoff, GW)], didx[gb],
                              isem[gb]).wait()

    def fill_d2d(gb):
        for k in range(G):
            for j in range(WIN // 16):
                d2d[gb][k, pl.ds(j * 16, 16)] = (
                    didx[gb][pl.ds(k * WIN + j * 16, 16)])

    for j in range(WIN // 16):
        d2d1[G - 2, pl.ds(j * 16, 16)] = pidx[pl.ds(j * 16, 16)]
        d2d1[G - 1, pl.ds(j * 16, 16)] = pidx[pl.ds(j * 16, 16)]
    for b in range(LAG):
        pltpu.async_copy(rows[b], agg_sh.at[pidx], ssem[b], add=True)
        if compute_cnt:
            pltpu.async_copy(ones_v, cnt_sh.at[pidx], csem[b], add=True)

    def pair_body(gg, carry):
        for gpar in range(2):
            gb = gpar
            g = 2 * gg + gpar
            geoff = ebase + g * GW
            wait_idx(gb, geoff)
            fill_d2d(gb)
            for k in range(G):
                b = k % NBUF
                bv = (k + LAG) % NBUF
                wait_scatter(b)
                pltpu.async_copy(
                    x_hbm.at[sidx[gb].at[pl.ds(k * WIN, WIN)]],
                    rows[b], gsem[b])
                wait_gather(bv)
                if k < LAG:
                    dv = d2d[gb ^ 1].at[G - LAG + k]
                else:
                    dv = d2d[gb].at[k - LAG]
                pltpu.async_copy(rows[bv], agg_sh.at[dv], ssem[bv], add=True)
                if compute_cnt:
                    pltpu.async_copy(ones_v, cnt_sh.at[dv], csem[bv],
                                     add=True)
                if k == 3:
                    nxt = ebase + (g + 1) * GW
                    pltpu.async_copy(src_hbm.at[pl.ds(nxt, GW)], sidx[gb ^ 1],
                                     isem[gb ^ 1])
                    pltpu.async_copy(dst_hbm.at[pl.ds(nxt, GW)], didx[gb ^ 1],
                                     isem[gb ^ 1])
        return carry

    lax.fori_loop(0, NG // 2, pair_body, 0)

    for e in range(LAG):
        kv = G - LAG + e
        bv = (WPW - LAG + e) % NBUF
        wait_gather(bv)
        pltpu.async_copy(rows[bv], agg_sh.at[d2d1.at[kv]], ssem[bv],
                         add=True)
        if compute_cnt:
            pltpu.async_copy(ones_v, cnt_sh.at[d2d1.at[kv]], csem[bv],
                             add=True)
    for b in range(NBUF):
        wait_scatter(b)
    wait_idx(0, ebase + NG * GW)

    plsc.subcore_barrier()

    for t in range(RPT // 128):
        pltpu.sync_copy(agg_sh.at[pl.ds(base + t * 128, 128)],
                        agg_out.at[c, pl.ds(base + t * 128, 128)])
    if compute_cnt:
        pltpu.sync_copy(cnt_sh.at[pl.ds(base, RPT)],
                        cnt_out.at[c, pl.ds(base, RPT)])


def _make_sc_agg(compute_cnt):
    mesh = plsc.VectorSubcoreMesh(core_axis_name="c", subcore_axis_name="s",
                                  num_cores=NC, num_subcores=NS)
    out_type = [jax.ShapeDtypeStruct((NC, N_PAD, D), jnp.float32)]
    if compute_cnt:
        out_type.append(jax.ShapeDtypeStruct((NC, N_PAD), jnp.float32))
    scratch = [
        pltpu.VMEM((GW,), jnp.int32),
        pltpu.VMEM((GW,), jnp.int32),
        pltpu.VMEM((GW,), jnp.int32),
        pltpu.VMEM((GW,), jnp.int32),
        pltpu.VMEM((G, WIN), jnp.int32),
        pltpu.VMEM((G, WIN), jnp.int32),
        pltpu.VMEM((WIN, D), jnp.float32),
        pltpu.VMEM((WIN, D), jnp.float32),
        pltpu.VMEM((WIN, D), jnp.float32),
        pltpu.VMEM((WIN, D), jnp.float32),
        pltpu.VMEM((WIN,), jnp.int32),
        pltpu.VMEM((WIN,), jnp.int32),
    ]
    if compute_cnt:
        scratch.append(pltpu.VMEM((WIN,), jnp.float32))
    scratch.append(pltpu.VMEM_SHARED((N_PAD, D), jnp.float32))
    if compute_cnt:
        scratch.append(pltpu.VMEM_SHARED((N_PAD,), jnp.float32))
    nsem = 10 + (4 if compute_cnt else 0)
    scratch += [pltpu.SemaphoreType.DMA] * nsem

    return pl.kernel(
        functools.partial(_sc_agg_body, compute_cnt),
        out_type=tuple(out_type),
        mesh=mesh,
        scratch_types=scratch,
        name=f"sage_sc_agg_cnt{int(compute_cnt)}",
    )


_SC_AGG_CNT = _make_sc_agg(True)
_SC_AGG = _make_sc_agg(False)


def _combine_body(relu, a, cc, xr, wn, ws, br, o):
    i = pl.program_id(0)
    cnt = (cc[0:1, pl.ds(i * _R_BLK, _R_BLK)]
           + cc[1:2, pl.ds(i * _R_BLK, _R_BLK)])
    r = 1.0 / jnp.maximum(cnt, 1.0)
    r_col = jnp.transpose(r, (1, 0))
    mean = (a[0] + a[1]) * r_col
    acc = jnp.dot(mean, wn[...], preferred_element_type=jnp.float32)
    acc = acc + jnp.dot(xr[...], ws[...], preferred_element_type=jnp.float32)
    acc = acc + br[...]
    if relu:
        acc = jnp.maximum(acc, 0.0)
    o[...] = acc


_R_BLK = 2048


def _combine(agg, cnt2, xr, wn, ws, br, relu):
    grid = (N_PAD // _R_BLK,)
    row_spec = pl.BlockSpec((_R_BLK, D), lambda i: (i, 0))
    a_spec = pl.BlockSpec((NC, _R_BLK, D), lambda i: (0, i, 0))
    c_spec = pl.BlockSpec((NC, N_PAD), lambda i: (0, 0))
    w_spec = pl.BlockSpec((D, D), lambda i: (0, 0))
    b_spec = pl.BlockSpec((1, D), lambda i: (0, 0))
    return pl.pallas_call(
        functools.partial(_combine_body, relu),
        grid=grid,
        in_specs=[a_spec, c_spec, row_spec, w_spec, w_spec, b_spec],
        out_specs=row_spec,
        out_shape=jax.ShapeDtypeStruct((N_NODES, D), jnp.float32),
        name=f"sage_combine_relu{int(relu)}",
    )(agg, cnt2, xr, wn, ws, br)


_PAD_N = E_PAD - N_EDGES
_PAD_SRC = jnp.asarray(np.arange(_PAD_N, dtype=np.int32) % N_NODES)
_PAD_DST = jnp.asarray(
    N_NODES + np.arange(_PAD_N, dtype=np.int32) % (N_PAD - N_NODES))


def kernel(x, edge_index, W1_self, W1_neigh, b1, W2_self, W2_neigh, b2):
    flat = lax.optimization_barrier(jnp.reshape(edge_index, (2 * N_EDGES,)))
    src_p = jnp.concatenate([flat[:N_EDGES], _PAD_SRC])
    dst_p = jnp.concatenate([flat[N_EDGES:], _PAD_DST])

    agg1, cnt = _SC_AGG_CNT(x, src_p, dst_p)
    b1r = b1.reshape(1, D)
    b2r = b2.reshape(1, D)

    h = _combine(agg1, cnt, x, W1_neigh, W1_self, b1r, relu=True)
    (agg2,) = _SC_AGG(h, src_p, dst_p)
    out = _combine(agg2, cnt, h, W2_neigh, W2_self, b2r, relu=False)
    return out
